# Initial kernel scaffold; baseline (speedup 1.0000x reference)
import jax
import jax.numpy as jnp
from jax.experimental import pallas as pl


def kernel(x, iso_type_2, Wi_r, Wi_n, bi, W00_r, W00_n, b00, W01_r, W01_n, b01, W10_r, W10_n, b10, W11_r, W11_n, b11, fc0_w, fc0_b, fc1_w, fc1_b, fc2_w, fc2_b, edge_index, edge_index_2, assignment_index_2, batch, batch_2):
    raise NotImplementedError("write your pallas kernel here")


# speedup vs baseline: 5.8671x; 5.8671x over previous
import functools
import math

import jax
import jax.numpy as jnp
from jax import lax
from jax.experimental import pallas as pl
from jax.experimental.pallas import tpu as pltpu
from jax.experimental.pallas import tpu_sc as plsc

_N, _E, _N2, _E2, _A, _B = 10000, 320000, 20000, 320000, 40000, 64
_D, _ISO, _NC = 128, 16, 10
_NSC = 2
_NSUB = 16
_W = 1000


def _elu(x):
    return jnp.where(x > 0, x, jnp.exp(jnp.minimum(x, 0.0)) - 1.0)


_BR = 1000


def _dot(a, b):
    return jnp.dot(a, b, preferred_element_type=jnp.float32)


def _k_init_body(h_ref, wa_ref, wb_ref, r_ref, t_ref):
    h = h_ref[...]
    r_ref[...] = _dot(h, wa_ref[...])
    t = _dot(h, wb_ref[...])
    t_ref[0] = t[:, :64]
    t_ref[1] = t[:, 64:]


def _k_step_body(r_ref, s_ref, b_ref, wa_ref, wb_ref, r2_ref, t2_ref):
    s = jnp.concatenate([s_ref[0], s_ref[1]], axis=1)
    h = _elu(r_ref[...] + s + b_ref[...])
    r2_ref[...] = _dot(h, wa_ref[...])
    t = _dot(h, wb_ref[...])
    t2_ref[0] = t[:, :64]
    t2_ref[1] = t[:, 64:]


def _k_fin_body(r_ref, s_ref, b_ref, o_ref):
    s = jnp.concatenate([s_ref[0], s_ref[1]], axis=1)
    h = _elu(r_ref[...] + s + b_ref[...])
    o_ref[0] = h[:, :64]
    o_ref[1] = h[:, 64:]


def _k_assign_body(ss_ref, cc_ref, iso_ref, wr_ref, wn_ref, r_ref, t_ref):
    cnt = jnp.maximum(cc_ref[0][:, :1] + cc_ref[1][:, :1], 1.0)
    m = jnp.concatenate([ss_ref[0], ss_ref[1]], axis=1) / cnt
    iso = iso_ref[...]
    r_ref[...] = _dot(m, wr_ref[:128]) + _dot(iso, wr_ref[128:])
    t = _dot(m, wn_ref[:128]) + _dot(iso, wn_ref[128:])
    t_ref[0] = t[:, :64]
    t_ref[1] = t[:, 64:]


def _k_head_body(p1s_ref, p1c_ref, p2s_ref, p2c_ref,
                 w0_ref, b0_ref, w1_ref, b1_ref, w2_ref, b2_ref, o_ref):
    c1 = jnp.maximum(p1c_ref[0][:, :1] + p1c_ref[1][:, :1], 1.0)
    p1 = jnp.concatenate([p1s_ref[0], p1s_ref[1]], axis=1) / c1
    c2 = jnp.maximum(p2c_ref[0][:, :1] + p2c_ref[1][:, :1], 1.0)
    p2 = jnp.concatenate([p2s_ref[0], p2s_ref[1]], axis=1) / c2
    g = jnp.concatenate([p1, p2], axis=1)
    g = _elu(_dot(g, w0_ref[...]) + b0_ref[...])
    g = _elu(_dot(g, w1_ref[...]) + b1_ref[...])
    g = _dot(g, w2_ref[...]) + b2_ref[...]
    m = jnp.max(g, axis=1, keepdims=True)
    z = g - m
    o_ref[...] = z - jnp.log(jnp.sum(jnp.exp(z), axis=1, keepdims=True))


def _row_spec(n):
    return pl.BlockSpec((_BR, n), lambda i: (i, 0))


def _split_spec():
    return pl.BlockSpec((2, _BR, 64), lambda i: (0, i, 0))


def _full_spec(shape):
    return pl.BlockSpec(shape, lambda i: tuple(0 for _ in shape))


def _tc_init(h, wa, wb):
    n = h.shape[0]
    return pl.pallas_call(
        _k_init_body,
        grid=(n // _BR,),
        in_specs=[_row_spec(128), _full_spec((128, 128)), _full_spec((128, 128))],
        out_specs=[_row_spec(128), _split_spec()],
        out_shape=[jax.ShapeDtypeStruct((n, 128), jnp.float32),
                   jax.ShapeDtypeStruct((2, n, 64), jnp.float32)],
    )(h, wa, wb)


def _tc_step(r, s, b, wa, wb):
    n = r.shape[0]
    return pl.pallas_call(
        _k_step_body,
        grid=(n // _BR,),
        in_specs=[_row_spec(128), _split_spec(), _full_spec((1, 128)),
                  _full_spec((128, 128)), _full_spec((128, 128))],
        out_specs=[_row_spec(128), _split_spec()],
        out_shape=[jax.ShapeDtypeStruct((n, 128), jnp.float32),
                   jax.ShapeDtypeStruct((2, n, 64), jnp.float32)],
    )(r, s, b.reshape(1, 128), wa, wb)


def _tc_fin(r, s, b):
    n = r.shape[0]
    return pl.pallas_call(
        _k_fin_body,
        grid=(n // _BR,),
        in_specs=[_row_spec(128), _split_spec(), _full_spec((1, 128))],
        out_specs=_split_spec(),
        out_shape=jax.ShapeDtypeStruct((2, n, 64), jnp.float32),
    )(r, s, b.reshape(1, 128))


def _tc_assign(ss, cc, iso, wr, wn):
    n = ss.shape[1]
    return pl.pallas_call(
        _k_assign_body,
        grid=(n // _BR,),
        in_specs=[_split_spec(),
                  pl.BlockSpec((2, _BR, 16), lambda i: (0, i, 0)),
                  _row_spec(16),
                  _full_spec((144, 128)), _full_spec((144, 128))],
        out_specs=[_row_spec(128), _split_spec()],
        out_shape=[jax.ShapeDtypeStruct((n, 128), jnp.float32),
                   jax.ShapeDtypeStruct((2, n, 64), jnp.float32)],
    )(ss, cc, iso, wr, wn)


def _tc_head(p1s, p1c, p2s, p2c, w0, b0, w1, b1, w2, b2):
    return pl.pallas_call(
        _k_head_body,
        grid=(1,),
        in_specs=[_full_spec((2, 64, 64)), _full_spec((2, 64, 16)),
                  _full_spec((2, 64, 64)), _full_spec((2, 64, 16)),
                  _full_spec((256, 128)), _full_spec((1, 128)),
                  _full_spec((128, 64)), _full_spec((1, 64)),
                  _full_spec((64, 10)), _full_spec((1, 10))],
        out_specs=_full_spec((64, 10)),
        out_shape=jax.ShapeDtypeStruct((64, 10), jnp.float32),
    )(p1s, p1c, p2s, p2c, w0, b0.reshape(1, 128), w1, b1.reshape(1, 64),
      w2, b2.reshape(1, 10))


def _zero_rows(zbuf, width, nrows):
    zv = jnp.zeros((16,), jnp.float32)

    def fill(i, _):
        for j in range(width // 16):
            zbuf[i, pl.ds(j * 16, 16)] = zv
        return 0
    lax.fori_loop(0, nrows, fill, 0)


def _rc_for(n_out):
    best = 8
    for cand in range(8, 257, 8):
        if n_out % cand == 0:
            best = cand
    return best


def _make_sc_segsum(n_in, n_out, n_idx, *, gather, counts, w):
    nw = n_idx // w
    assert nw * w == n_idx and w % 8 == 0
    nwmax = -(-nw // _NSUB)
    rc = min(_rc_for(n_out), w)
    nchunks = n_out // rc
    ncmax = -(-nchunks // _NSUB)
    mesh = plsc.VectorSubcoreMesh(core_axis_name="c", subcore_axis_name="s")

    out_type = [jax.ShapeDtypeStruct((2, n_out, 64), jnp.float32)]
    scratch = [
        pltpu.VMEM((w,), jnp.int32),
        pltpu.VMEM((w,), jnp.int32),
        pltpu.VMEM((w, 64), jnp.float32),
        pltpu.VMEM_SHARED((n_out, 64), jnp.float32),
    ]
    if counts:
        out_type.append(jax.ShapeDtypeStruct((2, n_out, 16), jnp.float32))
        scratch += [
            pltpu.VMEM((w, 16), jnp.float32),
            pltpu.VMEM_SHARED((n_out, 16), jnp.float32),
        ]

    def body(table, isrc, idst, *refs):
        if counts:
            out, ocnt, src_v, dst_v, rows_v, acc, ones_v, cacc = refs
        else:
            out, src_v, dst_v, rows_v, acc = refs
            ocnt = ones_v = cacc = None
        c = lax.axis_index("c")
        s = lax.axis_index("s")

        _zero_rows(rows_v, 64, rc)
        if counts:
            _zero_rows(ones_v, 16, rc)

        def zero_body(i, _):
            j = i * _NSUB + s

            @pl.when(j < nchunks)
            def _():
                pltpu.sync_copy(rows_v.at[pl.ds(0, rc)],
                                acc.at[pl.ds(j * rc, rc)])
                if counts:
                    pltpu.sync_copy(ones_v.at[pl.ds(0, rc)],
                                    cacc.at[pl.ds(j * rc, rc)])
            return 0
        lax.fori_loop(0, ncmax, zero_body, 0)

        if counts:
            ov = jnp.ones((16,), jnp.float32)

            def fill_ones(i, _):
                ones_v[i, pl.ds(0, 16)] = ov
                return 0
            lax.fori_loop(0, w, fill_ones, 0)
        plsc.subcore_barrier()

        def window_body(i, _):
            wi = i * _NSUB + s

            @pl.when(wi < nw)
            def _():
                base = wi * w
                pltpu.sync_copy(idst.at[pl.ds(base, w)], dst_v)
                if gather:
                    pltpu.sync_copy(isrc.at[pl.ds(base, w)], src_v)
                    pltpu.sync_copy(table.at[c].at[src_v], rows_v)
                else:
                    pltpu.sync_copy(table.at[c].at[pl.ds(base, w)], rows_v)
                pltpu.sync_copy(rows_v, acc.at[dst_v], add=True)
                if counts:
                    @pl.when((wi % 2) == c)
                    def _():
                        pltpu.sync_copy(ones_v, cacc.at[dst_v], add=True)
            return 0
        lax.fori_loop(0, nwmax, window_body, 0)
        plsc.subcore_barrier()

        def wb_body(i, _):
            j = i * _NSUB + s

            @pl.when(j < nchunks)
            def _():
                pltpu.sync_copy(acc.at[pl.ds(j * rc, rc)],
                                out.at[c].at[pl.ds(j * rc, rc)])
                if counts:
                    pltpu.sync_copy(cacc.at[pl.ds(j * rc, rc)],
                                    ocnt.at[c].at[pl.ds(j * rc, rc)])
            return 0
        lax.fori_loop(0, ncmax, wb_body, 0)

    return pl.kernel(body, out_type=out_type, mesh=mesh,
                     scratch_types=scratch,
                     compiler_params=pltpu.CompilerParams(
                         use_tc_tiling_on_sc=False))


_sc_edge1 = _make_sc_segsum(_N, _N, _E, gather=True, counts=False, w=400)
_sc_edge2 = _make_sc_segsum(_N2, _N2, _E2, gather=True, counts=False, w=400)
_sc_assign = _make_sc_segsum(_N, _N2, _A, gather=True, counts=True, w=200)
_sc_pool1 = _make_sc_segsum(_N, _B, _N, gather=False, counts=True, w=400)
_sc_pool2 = _make_sc_segsum(_N2, _B, _N2, gather=False, counts=True, w=400)


def kernel(x, iso_type_2, Wi_r, Wi_n, bi, W00_r, W00_n, b00, W01_r, W01_n, b01,
           W10_r, W10_n, b10, W11_r, W11_n, b11,
           fc0_w, fc0_b, fc1_w, fc1_b, fc2_w, fc2_b,
           edge_index, edge_index_2, assignment_index_2, batch, batch_2):
    src, dst = edge_index[0], edge_index[1]
    src2, dst2 = edge_index_2[0], edge_index_2[1]
    asrc, adst = assignment_index_2[0], assignment_index_2[1]
    lin1 = jnp.zeros((_N,), jnp.int32)
    lin2 = jnp.zeros((_N2,), jnp.int32)

    r0, t0 = _tc_init(x, Wi_r, Wi_n)
    (s0,) = _sc_edge1(t0, src, dst)
    r1, t1 = _tc_step(r0, s0, bi, W00_r, W00_n)
    (s1,) = _sc_edge1(t1, src, dst)
    r2, t2 = _tc_step(r1, s1, b00, W01_r, W01_n)
    (s2,) = _sc_edge1(t2, src, dst)
    hs = _tc_fin(r2, s2, b01)

    p1s, p1c = _sc_pool1(hs, lin1, batch)

    asum, acnt = _sc_assign(hs, asrc, adst)
    r3, t3 = _tc_assign(asum, acnt, iso_type_2, W10_r, W10_n)
    (s3,) = _sc_edge2(t3, src2, dst2)
    r4, t4 = _tc_step(r3, s3, b10, W11_r, W11_n)
    (s4,) = _sc_edge2(t4, src2, dst2)
    h2s = _tc_fin(r4, s4, b11)

    p2s, p2c = _sc_pool2(h2s, lin2, batch_2)

    return _tc_head(p1s, p1c, p2s, p2c, fc0_w, fc0_b, fc1_w, fc1_b,
                    fc2_w, fc2_b)


# speedup vs baseline: 7.7086x; 1.3139x over previous
import functools
import math

import jax
import jax.numpy as jnp
from jax import lax
from jax.experimental import pallas as pl
from jax.experimental.pallas import tpu as pltpu
from jax.experimental.pallas import tpu_sc as plsc

_N, _E, _N2, _E2, _A, _B = 10000, 320000, 20000, 320000, 40000, 64
_D, _ISO, _NC = 128, 16, 10
_NSC = 2
_NSUB = 16
_W = 1000


def _elu(x):
    return jnp.where(x > 0, x, jnp.exp(jnp.minimum(x, 0.0)) - 1.0)


_BR = 1000


def _dot(a, b):
    return jnp.dot(a, b, preferred_element_type=jnp.float32)


def _k_init_body(h_ref, wa_ref, wb_ref, r_ref, t_ref):
    h = h_ref[...]
    r_ref[...] = _dot(h, wa_ref[...])
    t = _dot(h, wb_ref[...])
    t_ref[0] = t[:, :64]
    t_ref[1] = t[:, 64:]


def _k_step_body(r_ref, s_ref, b_ref, wa_ref, wb_ref, r2_ref, t2_ref):
    s = jnp.concatenate([s_ref[0], s_ref[1]], axis=1)
    h = _elu(r_ref[...] + s + b_ref[...])
    r2_ref[...] = _dot(h, wa_ref[...])
    t = _dot(h, wb_ref[...])
    t2_ref[0] = t[:, :64]
    t2_ref[1] = t[:, 64:]


def _k_fin_body(r_ref, s_ref, b_ref, o_ref):
    s = jnp.concatenate([s_ref[0], s_ref[1]], axis=1)
    h = _elu(r_ref[...] + s + b_ref[...])
    o_ref[0] = h[:, :64]
    o_ref[1] = h[:, 64:]


def _k_assign_body(ss_ref, cc_ref, iso_ref, wr_ref, wn_ref, r_ref, t_ref):
    cnt = jnp.maximum(cc_ref[0][:, :1] + cc_ref[1][:, :1], 1.0)
    m = jnp.concatenate([ss_ref[0], ss_ref[1]], axis=1) / cnt
    iso = iso_ref[...]
    r_ref[...] = _dot(m, wr_ref[:128]) + _dot(iso, wr_ref[128:])
    t = _dot(m, wn_ref[:128]) + _dot(iso, wn_ref[128:])
    t_ref[0] = t[:, :64]
    t_ref[1] = t[:, 64:]


def _k_head_body(p1s_ref, p1c_ref, p2s_ref, p2c_ref,
                 w0_ref, b0_ref, w1_ref, b1_ref, w2_ref, b2_ref, o_ref):
    c1 = jnp.maximum(p1c_ref[0][:, :1] + p1c_ref[1][:, :1], 1.0)
    p1 = jnp.concatenate([p1s_ref[0], p1s_ref[1]], axis=1) / c1
    c2 = jnp.maximum(p2c_ref[0][:, :1] + p2c_ref[1][:, :1], 1.0)
    p2 = jnp.concatenate([p2s_ref[0], p2s_ref[1]], axis=1) / c2
    g = jnp.concatenate([p1, p2], axis=1)
    g = _elu(_dot(g, w0_ref[...]) + b0_ref[...])
    g = _elu(_dot(g, w1_ref[...]) + b1_ref[...])
    g = _dot(g, w2_ref[...]) + b2_ref[...]
    m = jnp.max(g, axis=1, keepdims=True)
    z = g - m
    o_ref[...] = z - jnp.log(jnp.sum(jnp.exp(z), axis=1, keepdims=True))


def _row_spec(n):
    return pl.BlockSpec((_BR, n), lambda i: (i, 0))


def _split_spec():
    return pl.BlockSpec((2, _BR, 64), lambda i: (0, i, 0))


def _full_spec(shape):
    return pl.BlockSpec(shape, lambda i: tuple(0 for _ in shape))


def _tc_init(h, wa, wb):
    n = h.shape[0]
    return pl.pallas_call(
        _k_init_body,
        grid=(n // _BR,),
        in_specs=[_row_spec(128), _full_spec((128, 128)), _full_spec((128, 128))],
        out_specs=[_row_spec(128), _split_spec()],
        out_shape=[jax.ShapeDtypeStruct((n, 128), jnp.float32),
                   jax.ShapeDtypeStruct((2, n, 64), jnp.float32)],
    )(h, wa, wb)


def _tc_step(r, s, b, wa, wb):
    n = r.shape[0]
    return pl.pallas_call(
        _k_step_body,
        grid=(n // _BR,),
        in_specs=[_row_spec(128), _split_spec(), _full_spec((1, 128)),
                  _full_spec((128, 128)), _full_spec((128, 128))],
        out_specs=[_row_spec(128), _split_spec()],
        out_shape=[jax.ShapeDtypeStruct((n, 128), jnp.float32),
                   jax.ShapeDtypeStruct((2, n, 64), jnp.float32)],
    )(r, s, b.reshape(1, 128), wa, wb)


def _tc_fin(r, s, b):
    n = r.shape[0]
    return pl.pallas_call(
        _k_fin_body,
        grid=(n // _BR,),
        in_specs=[_row_spec(128), _split_spec(), _full_spec((1, 128))],
        out_specs=_split_spec(),
        out_shape=jax.ShapeDtypeStruct((2, n, 64), jnp.float32),
    )(r, s, b.reshape(1, 128))


def _tc_assign(ss, cc, iso, wr, wn):
    n = ss.shape[1]
    return pl.pallas_call(
        _k_assign_body,
        grid=(n // _BR,),
        in_specs=[_split_spec(),
                  pl.BlockSpec((2, _BR, 16), lambda i: (0, i, 0)),
                  _row_spec(16),
                  _full_spec((144, 128)), _full_spec((144, 128))],
        out_specs=[_row_spec(128), _split_spec()],
        out_shape=[jax.ShapeDtypeStruct((n, 128), jnp.float32),
                   jax.ShapeDtypeStruct((2, n, 64), jnp.float32)],
    )(ss, cc, iso, wr, wn)


def _tc_head(p1s, p1c, p2s, p2c, w0, b0, w1, b1, w2, b2):
    return pl.pallas_call(
        _k_head_body,
        grid=(1,),
        in_specs=[_full_spec((2, 64, 64)), _full_spec((2, 64, 16)),
                  _full_spec((2, 64, 64)), _full_spec((2, 64, 16)),
                  _full_spec((256, 128)), _full_spec((1, 128)),
                  _full_spec((128, 64)), _full_spec((1, 64)),
                  _full_spec((64, 10)), _full_spec((1, 10))],
        out_specs=_full_spec((64, 10)),
        out_shape=jax.ShapeDtypeStruct((64, 10), jnp.float32),
    )(p1s, p1c, p2s, p2c, w0, b0.reshape(1, 128), w1, b1.reshape(1, 64),
      w2, b2.reshape(1, 10))


def _zero_rows(zbuf, width, nrows):
    zv = jnp.zeros((16,), jnp.float32)

    def fill(i, _):
        for j in range(width // 16):
            zbuf[i, pl.ds(j * 16, 16)] = zv
        return 0
    lax.fori_loop(0, nrows, fill, 0)


def _rc_for(n_out):
    best = 8
    for cand in range(8, 257, 8):
        if n_out % cand == 0:
            best = cand
    return best


def _make_sc_segsum(n_in, n_out, n_idx, *, gather, counts, w):
    nw = n_idx // w
    assert nw * w == n_idx and w % 8 == 0
    nwmax = -(-nw // _NSUB)
    rc = min(_rc_for(n_out), w)
    nchunks = n_out // rc
    ncmax = -(-nchunks // _NSUB)
    mesh = plsc.VectorSubcoreMesh(core_axis_name="c", subcore_axis_name="s")

    out_type = [jax.ShapeDtypeStruct((2, n_out, 64), jnp.float32)]
    scratch = [
        pltpu.VMEM((w,), jnp.int32),
        pltpu.VMEM((w,), jnp.int32),
        pltpu.VMEM((w, 64), jnp.float32),
        pltpu.VMEM_SHARED((n_out, 64), jnp.float32),
    ]
    if counts:
        out_type.append(jax.ShapeDtypeStruct((2, n_out, 16), jnp.float32))
        scratch += [
            pltpu.VMEM((w, 16), jnp.float32),
            pltpu.VMEM_SHARED((n_out, 16), jnp.float32),
        ]

    def body(table, isrc, idst, *refs):
        if counts:
            out, ocnt, src_v, dst_v, rows_v, acc, ones_v, cacc = refs
        else:
            out, src_v, dst_v, rows_v, acc = refs
            ocnt = ones_v = cacc = None
        c = lax.axis_index("c")
        s = lax.axis_index("s")

        _zero_rows(rows_v, 64, rc)
        if counts:
            _zero_rows(ones_v, 16, rc)

        def zero_body(i, _):
            j = i * _NSUB + s

            @pl.when(j < nchunks)
            def _():
                pltpu.sync_copy(rows_v.at[pl.ds(0, rc)],
                                acc.at[pl.ds(j * rc, rc)])
                if counts:
                    pltpu.sync_copy(ones_v.at[pl.ds(0, rc)],
                                    cacc.at[pl.ds(j * rc, rc)])
            return 0
        lax.fori_loop(0, ncmax, zero_body, 0)

        if counts:
            ov = jnp.ones((16,), jnp.float32)

            def fill_ones(i, _):
                ones_v[i, pl.ds(0, 16)] = ov
                return 0
            lax.fori_loop(0, w, fill_ones, 0)
        plsc.subcore_barrier()

        def window_body(i, _):
            wi = i * _NSUB + s

            @pl.when(wi < nw)
            def _():
                base = wi * w
                pltpu.sync_copy(idst.at[pl.ds(base, w)], dst_v)
                if gather:
                    pltpu.sync_copy(isrc.at[pl.ds(base, w)], src_v)
                    pltpu.sync_copy(table.at[c].at[src_v], rows_v)
                else:
                    pltpu.sync_copy(table.at[c].at[pl.ds(base, w)], rows_v)
                pltpu.sync_copy(rows_v, acc.at[dst_v], add=True)
                if counts:
                    @pl.when((wi % 2) == c)
                    def _():
                        pltpu.sync_copy(ones_v, cacc.at[dst_v], add=True)
            return 0
        lax.fori_loop(0, nwmax, window_body, 0)
        plsc.subcore_barrier()

        def wb_body(i, _):
            j = i * _NSUB + s

            @pl.when(j < nchunks)
            def _():
                pltpu.sync_copy(acc.at[pl.ds(j * rc, rc)],
                                out.at[c].at[pl.ds(j * rc, rc)])
                if counts:
                    pltpu.sync_copy(cacc.at[pl.ds(j * rc, rc)],
                                    ocnt.at[c].at[pl.ds(j * rc, rc)])
            return 0
        lax.fori_loop(0, ncmax, wb_body, 0)

    return pl.kernel(body, out_type=out_type, mesh=mesh,
                     scratch_types=scratch,
                     compiler_params=pltpu.CompilerParams(
                         use_tc_tiling_on_sc=False))


def _make_sc_edge_pipe(n_in, n_out, n_idx, w):
    nw = n_idx // w
    assert nw * w == n_idx and w % 8 == 0 and nw >= _NSUB
    trips = -(-nw // _NSUB)
    pairs = -(-trips // 2)
    rc = min(_rc_for(n_out), w)
    nchunks = n_out // rc
    ncmax = -(-nchunks // _NSUB)
    mesh = plsc.VectorSubcoreMesh(core_axis_name="c", subcore_axis_name="s")

    scratch = [
        pltpu.VMEM((w,), jnp.int32), pltpu.VMEM((w,), jnp.int32),
        pltpu.VMEM((w, 64), jnp.float32),
        pltpu.VMEM((w,), jnp.int32), pltpu.VMEM((w,), jnp.int32),
        pltpu.VMEM((w, 64), jnp.float32),
        pltpu.VMEM_SHARED((n_out, 64), jnp.float32),
        pltpu.SemaphoreType.DMA, pltpu.SemaphoreType.DMA,
    ]

    def body(table, isrc, idst, out, src0, dst0, rows0, src1, dst1, rows1,
             acc, sem0, sem1):
        c = lax.axis_index("c")
        s = lax.axis_index("s")

        _zero_rows(rows0, 64, rc)

        def zero_body(i, _):
            j = i * _NSUB + s

            @pl.when(j < nchunks)
            def _():
                pltpu.sync_copy(rows0.at[pl.ds(0, rc)],
                                acc.at[pl.ds(j * rc, rc)])
            return 0
        lax.fori_loop(0, ncmax, zero_body, 0)
        plsc.subcore_barrier()

        def load_and_start(j, srcb, dstb, rowsb, sem):
            base = (j * _NSUB + s) * w
            pltpu.sync_copy(idst.at[pl.ds(base, w)], dstb)
            pltpu.sync_copy(isrc.at[pl.ds(base, w)], srcb)
            pltpu.async_copy(table.at[c].at[srcb], rowsb, sem)

        def wait_and_scatter(srcb, dstb, rowsb, sem):
            pltpu.make_async_copy(table.at[c].at[srcb], rowsb, sem).wait()
            pltpu.sync_copy(rowsb, acc.at[dstb], add=True)

        load_and_start(0, src0, dst0, rows0, sem0)

        def pair_body(i, _):
            j0, j1, j2 = 2 * i, 2 * i + 1, 2 * i + 2

            @pl.when(j1 * _NSUB + s < nw)
            def _():
                load_and_start(j1, src1, dst1, rows1, sem1)

            @pl.when(j0 * _NSUB + s < nw)
            def _():
                wait_and_scatter(src0, dst0, rows0, sem0)

            @pl.when(j2 * _NSUB + s < nw)
            def _():
                load_and_start(j2, src0, dst0, rows0, sem0)

            @pl.when(j1 * _NSUB + s < nw)
            def _():
                wait_and_scatter(src1, dst1, rows1, sem1)
            return 0
        lax.fori_loop(0, pairs, pair_body, 0)
        plsc.subcore_barrier()

        def wb_body(i, _):
            j = i * _NSUB + s

            @pl.when(j < nchunks)
            def _():
                pltpu.sync_copy(acc.at[pl.ds(j * rc, rc)],
                                out.at[c].at[pl.ds(j * rc, rc)])
            return 0
        lax.fori_loop(0, ncmax, wb_body, 0)

    return pl.kernel(body,
                     out_type=[jax.ShapeDtypeStruct((2, n_out, 64),
                                                    jnp.float32)],
                     mesh=mesh, scratch_types=scratch,
                     compiler_params=pltpu.CompilerParams(
                         use_tc_tiling_on_sc=False))


_sc_edge1 = _make_sc_edge_pipe(_N, _N, _E, w=400)
_sc_edge2 = _make_sc_edge_pipe(_N2, _N2, _E2, w=320)
_sc_assign = _make_sc_segsum(_N, _N2, _A, gather=True, counts=True, w=200)
_sc_pool1 = _make_sc_segsum(_N, _B, _N, gather=False, counts=True, w=400)
_sc_pool2 = _make_sc_segsum(_N2, _B, _N2, gather=False, counts=True, w=400)


def kernel(x, iso_type_2, Wi_r, Wi_n, bi, W00_r, W00_n, b00, W01_r, W01_n, b01,
           W10_r, W10_n, b10, W11_r, W11_n, b11,
           fc0_w, fc0_b, fc1_w, fc1_b, fc2_w, fc2_b,
           edge_index, edge_index_2, assignment_index_2, batch, batch_2):
    src, dst = edge_index[0], edge_index[1]
    src2, dst2 = edge_index_2[0], edge_index_2[1]
    asrc, adst = assignment_index_2[0], assignment_index_2[1]
    lin1 = jnp.zeros((_N,), jnp.int32)
    lin2 = jnp.zeros((_N2,), jnp.int32)

    r0, t0 = _tc_init(x, Wi_r, Wi_n)
    (s0,) = _sc_edge1(t0, src, dst)
    r1, t1 = _tc_step(r0, s0, bi, W00_r, W00_n)
    (s1,) = _sc_edge1(t1, src, dst)
    r2, t2 = _tc_step(r1, s1, b00, W01_r, W01_n)
    (s2,) = _sc_edge1(t2, src, dst)
    hs = _tc_fin(r2, s2, b01)

    p1s, p1c = _sc_pool1(hs, lin1, batch)

    asum, acnt = _sc_assign(hs, asrc, adst)
    r3, t3 = _tc_assign(asum, acnt, iso_type_2, W10_r, W10_n)
    (s3,) = _sc_edge2(t3, src2, dst2)
    r4, t4 = _tc_step(r3, s3, b10, W11_r, W11_n)
    (s4,) = _sc_edge2(t4, src2, dst2)
    h2s = _tc_fin(r4, s4, b11)

    p2s, p2c = _sc_pool2(h2s, lin2, batch_2)

    return _tc_head(p1s, p1c, p2s, p2c, fc0_w, fc0_b, fc1_w, fc1_b,
                    fc2_w, fc2_b)


# speedup vs baseline: 7.8977x; 1.0245x over previous
import functools
import math

import jax
import jax.numpy as jnp
from jax import lax
from jax.experimental import pallas as pl
from jax.experimental.pallas import tpu as pltpu
from jax.experimental.pallas import tpu_sc as plsc

_N, _E, _N2, _E2, _A, _B = 10000, 320000, 20000, 320000, 40000, 64
_D, _ISO, _NC = 128, 16, 10
_NSC = 2
_NSUB = 16
_W = 1000


def _elu(x):
    return jnp.where(x > 0, x, jnp.exp(jnp.minimum(x, 0.0)) - 1.0)


_BR = 1000


def _dot(a, b):
    return jnp.dot(a, b, preferred_element_type=jnp.float32)


def _k_init_body(h_ref, wa_ref, wb_ref, r_ref, t_ref):
    h = h_ref[...]
    r_ref[...] = _dot(h, wa_ref[...])
    t_ref[...] = _dot(h, wb_ref[...])


def _k_step1_body(r_ref, s_ref, b_ref, wa_ref, wb_ref, r2_ref, t2_ref):
    h = _elu(r_ref[...] + s_ref[0] + s_ref[1] + b_ref[...])
    r2_ref[...] = _dot(h, wa_ref[...])
    t2_ref[...] = _dot(h, wb_ref[...])


def _k_fin1_body(r_ref, s_ref, b_ref, o_ref):
    h = _elu(r_ref[...] + s_ref[0] + s_ref[1] + b_ref[...])
    o_ref[0] = h[:, :64]
    o_ref[1] = h[:, 64:]


def _k_step_body(r_ref, s_ref, b_ref, wa_ref, wb_ref, r2_ref, t2_ref):
    s = jnp.concatenate([s_ref[0], s_ref[1]], axis=1)
    h = _elu(r_ref[...] + s + b_ref[...])
    r2_ref[...] = _dot(h, wa_ref[...])
    t = _dot(h, wb_ref[...])
    t2_ref[0] = t[:, :64]
    t2_ref[1] = t[:, 64:]


def _k_fin_body(r_ref, s_ref, b_ref, o_ref):
    s = jnp.concatenate([s_ref[0], s_ref[1]], axis=1)
    h = _elu(r_ref[...] + s + b_ref[...])
    o_ref[0] = h[:, :64]
    o_ref[1] = h[:, 64:]


def _k_assign_body(ss_ref, cc_ref, iso_ref, wr_ref, wn_ref, r_ref, t_ref):
    cnt = jnp.maximum(cc_ref[0][:, :1] + cc_ref[1][:, :1], 1.0)
    m = jnp.concatenate([ss_ref[0], ss_ref[1]], axis=1) / cnt
    iso = iso_ref[...]
    r_ref[...] = _dot(m, wr_ref[:128]) + _dot(iso, wr_ref[128:])
    t = _dot(m, wn_ref[:128]) + _dot(iso, wn_ref[128:])
    t_ref[0] = t[:, :64]
    t_ref[1] = t[:, 64:]


def _k_head_body(p1s_ref, p1c_ref, p2s_ref, p2c_ref,
                 w0_ref, b0_ref, w1_ref, b1_ref, w2_ref, b2_ref, o_ref):
    c1 = jnp.maximum(p1c_ref[0][:, :1] + p1c_ref[1][:, :1], 1.0)
    p1 = jnp.concatenate([p1s_ref[0], p1s_ref[1]], axis=1) / c1
    c2 = jnp.maximum(p2c_ref[0][:, :1] + p2c_ref[1][:, :1], 1.0)
    p2 = jnp.concatenate([p2s_ref[0], p2s_ref[1]], axis=1) / c2
    g = jnp.concatenate([p1, p2], axis=1)
    g = _elu(_dot(g, w0_ref[...]) + b0_ref[...])
    g = _elu(_dot(g, w1_ref[...]) + b1_ref[...])
    g = _dot(g, w2_ref[...]) + b2_ref[...]
    m = jnp.max(g, axis=1, keepdims=True)
    z = g - m
    o_ref[...] = z - jnp.log(jnp.sum(jnp.exp(z), axis=1, keepdims=True))


def _row_spec(n):
    return pl.BlockSpec((_BR, n), lambda i: (i, 0))


def _split_spec():
    return pl.BlockSpec((2, _BR, 64), lambda i: (0, i, 0))


def _full_spec(shape):
    return pl.BlockSpec(shape, lambda i: tuple(0 for _ in shape))


def _part_spec():
    return pl.BlockSpec((2, _BR, 128), lambda i: (0, i, 0))


def _tc_init(h, wa, wb):
    n = h.shape[0]
    return pl.pallas_call(
        _k_init_body,
        grid=(n // _BR,),
        in_specs=[_row_spec(128), _full_spec((128, 128)), _full_spec((128, 128))],
        out_specs=[_row_spec(128), _row_spec(128)],
        out_shape=[jax.ShapeDtypeStruct((n, 128), jnp.float32),
                   jax.ShapeDtypeStruct((n, 128), jnp.float32)],
    )(h, wa, wb)


def _tc_step1(r, s, b, wa, wb):
    n = r.shape[0]
    return pl.pallas_call(
        _k_step1_body,
        grid=(n // _BR,),
        in_specs=[_row_spec(128), _part_spec(), _full_spec((1, 128)),
                  _full_spec((128, 128)), _full_spec((128, 128))],
        out_specs=[_row_spec(128), _row_spec(128)],
        out_shape=[jax.ShapeDtypeStruct((n, 128), jnp.float32),
                   jax.ShapeDtypeStruct((n, 128), jnp.float32)],
    )(r, s, b.reshape(1, 128), wa, wb)


def _tc_fin1(r, s, b):
    n = r.shape[0]
    return pl.pallas_call(
        _k_fin1_body,
        grid=(n // _BR,),
        in_specs=[_row_spec(128), _part_spec(), _full_spec((1, 128))],
        out_specs=_split_spec(),
        out_shape=jax.ShapeDtypeStruct((2, n, 64), jnp.float32),
    )(r, s, b.reshape(1, 128))


def _tc_step(r, s, b, wa, wb):
    n = r.shape[0]
    return pl.pallas_call(
        _k_step_body,
        grid=(n // _BR,),
        in_specs=[_row_spec(128), _split_spec(), _full_spec((1, 128)),
                  _full_spec((128, 128)), _full_spec((128, 128))],
        out_specs=[_row_spec(128), _split_spec()],
        out_shape=[jax.ShapeDtypeStruct((n, 128), jnp.float32),
                   jax.ShapeDtypeStruct((2, n, 64), jnp.float32)],
    )(r, s, b.reshape(1, 128), wa, wb)


def _tc_fin(r, s, b):
    n = r.shape[0]
    return pl.pallas_call(
        _k_fin_body,
        grid=(n // _BR,),
        in_specs=[_row_spec(128), _split_spec(), _full_spec((1, 128))],
        out_specs=_split_spec(),
        out_shape=jax.ShapeDtypeStruct((2, n, 64), jnp.float32),
    )(r, s, b.reshape(1, 128))


def _tc_assign(ss, cc, iso, wr, wn):
    n = ss.shape[1]
    return pl.pallas_call(
        _k_assign_body,
        grid=(n // _BR,),
        in_specs=[_split_spec(),
                  pl.BlockSpec((2, _BR, 16), lambda i: (0, i, 0)),
                  _row_spec(16),
                  _full_spec((144, 128)), _full_spec((144, 128))],
        out_specs=[_row_spec(128), _split_spec()],
        out_shape=[jax.ShapeDtypeStruct((n, 128), jnp.float32),
                   jax.ShapeDtypeStruct((2, n, 64), jnp.float32)],
    )(ss, cc, iso, wr, wn)


def _tc_head(p1s, p1c, p2s, p2c, w0, b0, w1, b1, w2, b2):
    return pl.pallas_call(
        _k_head_body,
        grid=(1,),
        in_specs=[_full_spec((2, 64, 64)), _full_spec((2, 64, 16)),
                  _full_spec((2, 64, 64)), _full_spec((2, 64, 16)),
                  _full_spec((256, 128)), _full_spec((1, 128)),
                  _full_spec((128, 64)), _full_spec((1, 64)),
                  _full_spec((64, 10)), _full_spec((1, 10))],
        out_specs=_full_spec((64, 10)),
        out_shape=jax.ShapeDtypeStruct((64, 10), jnp.float32),
    )(p1s, p1c, p2s, p2c, w0, b0.reshape(1, 128), w1, b1.reshape(1, 64),
      w2, b2.reshape(1, 10))


def _zero_rows(zbuf, width, nrows):
    zv = jnp.zeros((16,), jnp.float32)

    def fill(i, _):
        for j in range(width // 16):
            zbuf[i, pl.ds(j * 16, 16)] = zv
        return 0
    lax.fori_loop(0, nrows, fill, 0)


def _rc_for(n_out, cap=256):
    best = 8
    for cand in range(8, cap + 1, 8):
        if n_out % cand == 0:
            best = cand
    return best


def _make_sc_segsum(n_in, n_out, n_idx, *, gather, counts, w):
    nw = n_idx // w
    assert nw * w == n_idx and w % 8 == 0
    nwmax = -(-nw // _NSUB)
    rc = _rc_for(n_out, cap=min(w, 256))
    nchunks = n_out // rc
    ncmax = -(-nchunks // _NSUB)
    mesh = plsc.VectorSubcoreMesh(core_axis_name="c", subcore_axis_name="s")

    out_type = [jax.ShapeDtypeStruct((2, n_out, 64), jnp.float32)]
    scratch = [
        pltpu.VMEM((w,), jnp.int32),
        pltpu.VMEM((w,), jnp.int32),
        pltpu.VMEM((w, 64), jnp.float32),
        pltpu.VMEM_SHARED((n_out, 64), jnp.float32),
    ]
    if counts:
        out_type.append(jax.ShapeDtypeStruct((2, n_out, 16), jnp.float32))
        scratch += [
            pltpu.VMEM((w, 16), jnp.float32),
            pltpu.VMEM_SHARED((n_out, 16), jnp.float32),
        ]

    def body(table, isrc, idst, *refs):
        if counts:
            out, ocnt, src_v, dst_v, rows_v, acc, ones_v, cacc = refs
        else:
            out, src_v, dst_v, rows_v, acc = refs
            ocnt = ones_v = cacc = None
        c = lax.axis_index("c")
        s = lax.axis_index("s")

        _zero_rows(rows_v, 64, rc)
        if counts:
            _zero_rows(ones_v, 16, rc)

        def zero_body(i, _):
            j = i * _NSUB + s

            @pl.when(j < nchunks)
            def _():
                pltpu.sync_copy(rows_v.at[pl.ds(0, rc)],
                                acc.at[pl.ds(j * rc, rc)])
                if counts:
                    pltpu.sync_copy(ones_v.at[pl.ds(0, rc)],
                                    cacc.at[pl.ds(j * rc, rc)])
            return 0
        lax.fori_loop(0, ncmax, zero_body, 0)

        if counts:
            ov = jnp.ones((16,), jnp.float32)

            def fill_ones(i, _):
                ones_v[i, pl.ds(0, 16)] = ov
                return 0
            lax.fori_loop(0, w, fill_ones, 0)
        plsc.subcore_barrier()

        def window_body(i, _):
            wi = i * _NSUB + s

            @pl.when(wi < nw)
            def _():
                base = wi * w
                pltpu.sync_copy(idst.at[pl.ds(base, w)], dst_v)
                if gather:
                    pltpu.sync_copy(isrc.at[pl.ds(base, w)], src_v)
                    pltpu.sync_copy(table.at[c].at[src_v], rows_v)
                else:
                    pltpu.sync_copy(table.at[c].at[pl.ds(base, w)], rows_v)
                pltpu.sync_copy(rows_v, acc.at[dst_v], add=True)
                if counts:
                    @pl.when((wi % 2) == c)
                    def _():
                        pltpu.sync_copy(ones_v, cacc.at[dst_v], add=True)
            return 0
        lax.fori_loop(0, nwmax, window_body, 0)
        plsc.subcore_barrier()

        def wb_body(i, _):
            j = i * _NSUB + s

            @pl.when(j < nchunks)
            def _():
                pltpu.sync_copy(acc.at[pl.ds(j * rc, rc)],
                                out.at[c].at[pl.ds(j * rc, rc)])
                if counts:
                    pltpu.sync_copy(cacc.at[pl.ds(j * rc, rc)],
                                    ocnt.at[c].at[pl.ds(j * rc, rc)])
            return 0
        lax.fori_loop(0, ncmax, wb_body, 0)

    return pl.kernel(body, out_type=out_type, mesh=mesh,
                     scratch_types=scratch,
                     compiler_params=pltpu.CompilerParams(
                         use_tc_tiling_on_sc=False))


def _make_sc_edge_pipe(n_in, n_out, n_idx, w):
    nw = n_idx // w
    assert nw * w == n_idx and w % 8 == 0 and nw >= _NSUB
    trips = -(-nw // _NSUB)
    pairs = -(-trips // 2)
    rc = _rc_for(n_out, cap=min(w, 256))
    nchunks = n_out // rc
    ncmax = -(-nchunks // _NSUB)
    mesh = plsc.VectorSubcoreMesh(core_axis_name="c", subcore_axis_name="s")

    scratch = [
        pltpu.VMEM((w,), jnp.int32), pltpu.VMEM((w,), jnp.int32),
        pltpu.VMEM((w, 64), jnp.float32),
        pltpu.VMEM((w,), jnp.int32), pltpu.VMEM((w,), jnp.int32),
        pltpu.VMEM((w, 64), jnp.float32),
        pltpu.VMEM_SHARED((n_out, 64), jnp.float32),
        pltpu.SemaphoreType.DMA, pltpu.SemaphoreType.DMA,
    ]

    def body(table, isrc, idst, out, src0, dst0, rows0, src1, dst1, rows1,
             acc, sem0, sem1):
        c = lax.axis_index("c")
        s = lax.axis_index("s")

        _zero_rows(rows0, 64, rc)

        def zero_body(i, _):
            j = i * _NSUB + s

            @pl.when(j < nchunks)
            def _():
                pltpu.sync_copy(rows0.at[pl.ds(0, rc)],
                                acc.at[pl.ds(j * rc, rc)])
            return 0
        lax.fori_loop(0, ncmax, zero_body, 0)
        plsc.subcore_barrier()

        def load_and_start(j, srcb, dstb, rowsb, sem):
            base = (j * _NSUB + s) * w
            pltpu.sync_copy(idst.at[pl.ds(base, w)], dstb)
            pltpu.sync_copy(isrc.at[pl.ds(base, w)], srcb)
            pltpu.async_copy(table.at[c].at[srcb], rowsb, sem)

        def wait_and_scatter(srcb, dstb, rowsb, sem):
            pltpu.make_async_copy(table.at[c].at[srcb], rowsb, sem).wait()
            pltpu.sync_copy(rowsb, acc.at[dstb], add=True)

        load_and_start(0, src0, dst0, rows0, sem0)

        def pair_body(i, _):
            j0, j1, j2 = 2 * i, 2 * i + 1, 2 * i + 2

            @pl.when(j1 * _NSUB + s < nw)
            def _():
                load_and_start(j1, src1, dst1, rows1, sem1)

            @pl.when(j0 * _NSUB + s < nw)
            def _():
                wait_and_scatter(src0, dst0, rows0, sem0)

            @pl.when(j2 * _NSUB + s < nw)
            def _():
                load_and_start(j2, src0, dst0, rows0, sem0)

            @pl.when(j1 * _NSUB + s < nw)
            def _():
                wait_and_scatter(src1, dst1, rows1, sem1)
            return 0
        lax.fori_loop(0, pairs, pair_body, 0)
        plsc.subcore_barrier()

        def wb_body(i, _):
            j = i * _NSUB + s

            @pl.when(j < nchunks)
            def _():
                pltpu.sync_copy(acc.at[pl.ds(j * rc, rc)],
                                out.at[c].at[pl.ds(j * rc, rc)])
            return 0
        lax.fori_loop(0, ncmax, wb_body, 0)

    return pl.kernel(body,
                     out_type=[jax.ShapeDtypeStruct((2, n_out, 64),
                                                    jnp.float32)],
                     mesh=mesh, scratch_types=scratch,
                     compiler_params=pltpu.CompilerParams(
                         use_tc_tiling_on_sc=False))


def _make_sc_edge_full(n, n_idx, w):
    nw = n_idx // w
    assert nw * w == n_idx and w % 8 == 0
    trips = -(-nw // (2 * _NSUB))
    pairs = -(-trips // 2)
    rc = _rc_for(n, cap=min(w, 256))
    nchunks = n // rc
    ncmax = -(-nchunks // _NSUB)
    mesh = plsc.VectorSubcoreMesh(core_axis_name="c", subcore_axis_name="s")

    scratch = [
        pltpu.VMEM((w,), jnp.int32), pltpu.VMEM((w,), jnp.int32),
        pltpu.VMEM((w, 128), jnp.float32),
        pltpu.VMEM((w,), jnp.int32), pltpu.VMEM((w,), jnp.int32),
        pltpu.VMEM((w, 128), jnp.float32),
        pltpu.VMEM_SHARED((n, 128), jnp.float32),
        pltpu.SemaphoreType.DMA, pltpu.SemaphoreType.DMA,
    ]

    def body(table, isrc, idst, out, src0, dst0, rows0, src1, dst1, rows1,
             acc, sem0, sem1):
        c = lax.axis_index("c")
        s = lax.axis_index("s")

        _zero_rows(rows0, 128, rc)

        def zero_body(i, _):
            j = i * _NSUB + s

            @pl.when(j < nchunks)
            def _():
                pltpu.sync_copy(rows0.at[pl.ds(0, rc)],
                                acc.at[pl.ds(j * rc, rc)])
            return 0
        lax.fori_loop(0, ncmax, zero_body, 0)
        plsc.subcore_barrier()

        def wi_of(j):
            return (j * _NSUB + s) * 2 + c

        def load_and_start(j, srcb, dstb, rowsb, sem):
            base = wi_of(j) * w
            pltpu.sync_copy(idst.at[pl.ds(base, w)], dstb)
            pltpu.sync_copy(isrc.at[pl.ds(base, w)], srcb)
            pltpu.async_copy(table.at[srcb], rowsb, sem)

        def wait_and_scatter(srcb, dstb, rowsb, sem):
            pltpu.make_async_copy(table.at[srcb], rowsb, sem).wait()
            pltpu.sync_copy(rowsb, acc.at[dstb], add=True)

        @pl.when(wi_of(0) < nw)
        def _():
            load_and_start(0, src0, dst0, rows0, sem0)

        def pair_body(i, _):
            j0, j1, j2 = 2 * i, 2 * i + 1, 2 * i + 2

            @pl.when(wi_of(j1) < nw)
            def _():
                load_and_start(j1, src1, dst1, rows1, sem1)

            @pl.when(wi_of(j0) < nw)
            def _():
                wait_and_scatter(src0, dst0, rows0, sem0)

            @pl.when(wi_of(j2) < nw)
            def _():
                load_and_start(j2, src0, dst0, rows0, sem0)

            @pl.when(wi_of(j1) < nw)
            def _():
                wait_and_scatter(src1, dst1, rows1, sem1)
            return 0
        lax.fori_loop(0, pairs, pair_body, 0)
        plsc.subcore_barrier()

        def wb_body(i, _):
            j = i * _NSUB + s

            @pl.when(j < nchunks)
            def _():
                pltpu.sync_copy(acc.at[pl.ds(j * rc, rc)],
                                out.at[c].at[pl.ds(j * rc, rc)])
            return 0
        lax.fori_loop(0, ncmax, wb_body, 0)

    return pl.kernel(body,
                     out_type=[jax.ShapeDtypeStruct((2, n, 128),
                                                    jnp.float32)],
                     mesh=mesh, scratch_types=scratch,
                     compiler_params=pltpu.CompilerParams(
                         use_tc_tiling_on_sc=True))


_sc_edge1 = _make_sc_edge_full(_N, _E, w=160)
_sc_edge2 = _make_sc_edge_pipe(_N2, _N2, _E2, w=320)
_sc_assign = _make_sc_segsum(_N, _N2, _A, gather=True, counts=True, w=200)
_sc_pool1 = _make_sc_segsum(_N, _B, _N, gather=False, counts=True, w=400)
_sc_pool2 = _make_sc_segsum(_N2, _B, _N2, gather=False, counts=True, w=400)


def kernel(x, iso_type_2, Wi_r, Wi_n, bi, W00_r, W00_n, b00, W01_r, W01_n, b01,
           W10_r, W10_n, b10, W11_r, W11_n, b11,
           fc0_w, fc0_b, fc1_w, fc1_b, fc2_w, fc2_b,
           edge_index, edge_index_2, assignment_index_2, batch, batch_2):
    src, dst = edge_index[0], edge_index[1]
    src2, dst2 = edge_index_2[0], edge_index_2[1]
    asrc, adst = assignment_index_2[0], assignment_index_2[1]
    lin1 = jnp.zeros((_N,), jnp.int32)
    lin2 = jnp.zeros((_N2,), jnp.int32)

    r0, t0 = _tc_init(x, Wi_r, Wi_n)
    (s0,) = _sc_edge1(t0, src, dst)
    r1, t1 = _tc_step1(r0, s0, bi, W00_r, W00_n)
    (s1,) = _sc_edge1(t1, src, dst)
    r2, t2 = _tc_step1(r1, s1, b00, W01_r, W01_n)
    (s2,) = _sc_edge1(t2, src, dst)
    hs = _tc_fin1(r2, s2, b01)

    p1s, p1c = _sc_pool1(hs, lin1, batch)

    asum, acnt = _sc_assign(hs, asrc, adst)
    r3, t3 = _tc_assign(asum, acnt, iso_type_2, W10_r, W10_n)
    (s3,) = _sc_edge2(t3, src2, dst2)
    r4, t4 = _tc_step(r3, s3, b10, W11_r, W11_n)
    (s4,) = _sc_edge2(t4, src2, dst2)
    h2s = _tc_fin(r4, s4, b11)

    p2s, p2c = _sc_pool2(h2s, lin2, batch_2)

    return _tc_head(p1s, p1c, p2s, p2c, fc0_w, fc0_b, fc1_w, fc1_b,
                    fc2_w, fc2_b)


# speedup vs baseline: 8.8357x; 1.1188x over previous
import functools
import math

import jax
import jax.numpy as jnp
from jax import lax
from jax.experimental import pallas as pl
from jax.experimental.pallas import tpu as pltpu
from jax.experimental.pallas import tpu_sc as plsc

_N, _E, _N2, _E2, _A, _B = 10000, 320000, 20000, 320000, 40000, 64
_D, _ISO, _NC = 128, 16, 10
_NSC = 2
_NSUB = 16
_W = 1000


def _elu(x):
    return jnp.where(x > 0, x, jnp.exp(jnp.minimum(x, 0.0)) - 1.0)


_BR = 1000


def _dot(a, b):
    return jnp.dot(a, b, preferred_element_type=jnp.float32)


def _k_init_body(h_ref, wa_ref, wb_ref, r_ref, t_ref):
    h = h_ref[...]
    r_ref[...] = _dot(h, wa_ref[...])
    t_ref[...] = _dot(h, wb_ref[...])


def _k_step1_body(r_ref, s_ref, b_ref, wa_ref, wb_ref, r2_ref, t2_ref):
    h = _elu(r_ref[...] + s_ref[0] + s_ref[1] + b_ref[...])
    r2_ref[...] = _dot(h, wa_ref[...])
    t2_ref[...] = _dot(h, wb_ref[...])


def _k_fin1_body(r_ref, s_ref, b_ref, o_ref):
    h = _elu(r_ref[...] + s_ref[0] + s_ref[1] + b_ref[...])
    o_ref[0] = h[:, :64]
    o_ref[1] = h[:, 64:]


def _k_step_body(r_ref, s_ref, b_ref, wa_ref, wb_ref, r2_ref, t2_ref):
    s = jnp.concatenate([s_ref[0], s_ref[1]], axis=1)
    h = _elu(r_ref[...] + s + b_ref[...])
    r2_ref[...] = _dot(h, wa_ref[...])
    t = _dot(h, wb_ref[...])
    t2_ref[0] = t[:, :64]
    t2_ref[1] = t[:, 64:]


def _k_fin_body(r_ref, s_ref, b_ref, o_ref):
    s = jnp.concatenate([s_ref[0], s_ref[1]], axis=1)
    h = _elu(r_ref[...] + s + b_ref[...])
    o_ref[0] = h[:, :64]
    o_ref[1] = h[:, 64:]


def _k_assign_body(ss_ref, cc_ref, iso_ref, wr_ref, wn_ref, r_ref, t_ref):
    cnt = jnp.maximum(cc_ref[0][:, :1] + cc_ref[1][:, :1], 1.0)
    m = jnp.concatenate([ss_ref[0], ss_ref[1]], axis=1) / cnt
    iso = iso_ref[...]
    r_ref[...] = _dot(m, wr_ref[:128]) + _dot(iso, wr_ref[128:])
    t = _dot(m, wn_ref[:128]) + _dot(iso, wn_ref[128:])
    t_ref[0] = t[:, :64]
    t_ref[1] = t[:, 64:]


def _k_head_body(p1s_ref, p1c_ref, p2s_ref, p2c_ref,
                 w0_ref, b0_ref, w1_ref, b1_ref, w2_ref, b2_ref, o_ref):
    c1 = jnp.maximum(p1c_ref[0][:, :1] + p1c_ref[1][:, :1], 1.0)
    p1 = jnp.concatenate([p1s_ref[0], p1s_ref[1]], axis=1) / c1
    c2 = jnp.maximum(p2c_ref[0][:, :1] + p2c_ref[1][:, :1], 1.0)
    p2 = jnp.concatenate([p2s_ref[0], p2s_ref[1]], axis=1) / c2
    g = jnp.concatenate([p1, p2], axis=1)
    g = _elu(_dot(g, w0_ref[...]) + b0_ref[...])
    g = _elu(_dot(g, w1_ref[...]) + b1_ref[...])
    g = _dot(g, w2_ref[...]) + b2_ref[...]
    m = jnp.max(g, axis=1, keepdims=True)
    z = g - m
    o_ref[...] = z - jnp.log(jnp.sum(jnp.exp(z), axis=1, keepdims=True))


def _row_spec(n):
    return pl.BlockSpec((_BR, n), lambda i: (i, 0))


def _split_spec():
    return pl.BlockSpec((2, _BR, 64), lambda i: (0, i, 0))


def _full_spec(shape):
    return pl.BlockSpec(shape, lambda i: tuple(0 for _ in shape))


def _part_spec():
    return pl.BlockSpec((2, _BR, 128), lambda i: (0, i, 0))


def _tc_init(h, wa, wb):
    n = h.shape[0]
    return pl.pallas_call(
        _k_init_body,
        grid=(n // _BR,),
        in_specs=[_row_spec(128), _full_spec((128, 128)), _full_spec((128, 128))],
        out_specs=[_row_spec(128), _row_spec(128)],
        out_shape=[jax.ShapeDtypeStruct((n, 128), jnp.float32),
                   jax.ShapeDtypeStruct((n, 128), jnp.float32)],
    )(h, wa, wb)


def _tc_step1(r, s, b, wa, wb):
    n = r.shape[0]
    return pl.pallas_call(
        _k_step1_body,
        grid=(n // _BR,),
        in_specs=[_row_spec(128), _part_spec(), _full_spec((1, 128)),
                  _full_spec((128, 128)), _full_spec((128, 128))],
        out_specs=[_row_spec(128), _row_spec(128)],
        out_shape=[jax.ShapeDtypeStruct((n, 128), jnp.float32),
                   jax.ShapeDtypeStruct((n, 128), jnp.float32)],
    )(r, s, b.reshape(1, 128), wa, wb)


def _tc_fin1(r, s, b):
    n = r.shape[0]
    return pl.pallas_call(
        _k_fin1_body,
        grid=(n // _BR,),
        in_specs=[_row_spec(128), _part_spec(), _full_spec((1, 128))],
        out_specs=_split_spec(),
        out_shape=jax.ShapeDtypeStruct((2, n, 64), jnp.float32),
    )(r, s, b.reshape(1, 128))


def _tc_step(r, s, b, wa, wb):
    n = r.shape[0]
    return pl.pallas_call(
        _k_step_body,
        grid=(n // _BR,),
        in_specs=[_row_spec(128), _split_spec(), _full_spec((1, 128)),
                  _full_spec((128, 128)), _full_spec((128, 128))],
        out_specs=[_row_spec(128), _split_spec()],
        out_shape=[jax.ShapeDtypeStruct((n, 128), jnp.float32),
                   jax.ShapeDtypeStruct((2, n, 64), jnp.float32)],
    )(r, s, b.reshape(1, 128), wa, wb)


def _tc_fin(r, s, b):
    n = r.shape[0]
    return pl.pallas_call(
        _k_fin_body,
        grid=(n // _BR,),
        in_specs=[_row_spec(128), _split_spec(), _full_spec((1, 128))],
        out_specs=_split_spec(),
        out_shape=jax.ShapeDtypeStruct((2, n, 64), jnp.float32),
    )(r, s, b.reshape(1, 128))


def _tc_assign(ss, cc, iso, wr, wn):
    n = ss.shape[1]
    return pl.pallas_call(
        _k_assign_body,
        grid=(n // _BR,),
        in_specs=[_split_spec(),
                  pl.BlockSpec((2, _BR, 16), lambda i: (0, i, 0)),
                  _row_spec(16),
                  _full_spec((144, 128)), _full_spec((144, 128))],
        out_specs=[_row_spec(128), _split_spec()],
        out_shape=[jax.ShapeDtypeStruct((n, 128), jnp.float32),
                   jax.ShapeDtypeStruct((2, n, 64), jnp.float32)],
    )(ss, cc, iso, wr, wn)


def _tc_head(p1s, p1c, p2s, p2c, w0, b0, w1, b1, w2, b2):
    return pl.pallas_call(
        _k_head_body,
        grid=(1,),
        in_specs=[_full_spec((2, 64, 64)), _full_spec((2, 64, 16)),
                  _full_spec((2, 64, 64)), _full_spec((2, 64, 16)),
                  _full_spec((256, 128)), _full_spec((1, 128)),
                  _full_spec((128, 64)), _full_spec((1, 64)),
                  _full_spec((64, 10)), _full_spec((1, 10))],
        out_specs=_full_spec((64, 10)),
        out_shape=jax.ShapeDtypeStruct((64, 10), jnp.float32),
    )(p1s, p1c, p2s, p2c, w0, b0.reshape(1, 128), w1, b1.reshape(1, 64),
      w2, b2.reshape(1, 10))


def _zero_rows(zbuf, width, nrows):
    zv = jnp.zeros((16,), jnp.float32)

    def fill(i, _):
        for j in range(width // 16):
            zbuf[i, pl.ds(j * 16, 16)] = zv
        return 0
    lax.fori_loop(0, nrows, fill, 0)


def _rc_for(n_out, cap=256):
    best = 8
    for cand in range(8, cap + 1, 8):
        if n_out % cand == 0:
            best = cand
    return best


def _make_sc_segsum(n_in, n_out, n_idx, *, gather, counts, w):
    nw = n_idx // w
    assert nw * w == n_idx and w % 8 == 0
    nwmax = -(-nw // _NSUB)
    rc = _rc_for(n_out, cap=min(w, 256))
    nchunks = n_out // rc
    ncmax = -(-nchunks // _NSUB)
    mesh = plsc.VectorSubcoreMesh(core_axis_name="c", subcore_axis_name="s")

    out_type = [jax.ShapeDtypeStruct((2, n_out, 64), jnp.float32)]
    scratch = [
        pltpu.VMEM((w,), jnp.int32),
        pltpu.VMEM((w,), jnp.int32),
        pltpu.VMEM((w, 64), jnp.float32),
        pltpu.VMEM_SHARED((n_out, 64), jnp.float32),
    ]
    if counts:
        out_type.append(jax.ShapeDtypeStruct((2, n_out, 16), jnp.float32))
        scratch += [
            pltpu.VMEM((w, 16), jnp.float32),
            pltpu.VMEM_SHARED((n_out, 16), jnp.float32),
        ]

    def body(table, isrc, idst, *refs):
        if counts:
            out, ocnt, src_v, dst_v, rows_v, acc, ones_v, cacc = refs
        else:
            out, src_v, dst_v, rows_v, acc = refs
            ocnt = ones_v = cacc = None
        c = lax.axis_index("c")
        s = lax.axis_index("s")

        _zero_rows(rows_v, 64, rc)
        if counts:
            _zero_rows(ones_v, 16, rc)

        def zero_body(i, _):
            j = i * _NSUB + s

            @pl.when(j < nchunks)
            def _():
                pltpu.sync_copy(rows_v.at[pl.ds(0, rc)],
                                acc.at[pl.ds(j * rc, rc)])
                if counts:
                    pltpu.sync_copy(ones_v.at[pl.ds(0, rc)],
                                    cacc.at[pl.ds(j * rc, rc)])
            return 0
        lax.fori_loop(0, ncmax, zero_body, 0)

        if counts:
            ov = jnp.ones((16,), jnp.float32)

            def fill_ones(i, _):
                ones_v[i, pl.ds(0, 16)] = ov
                return 0
            lax.fori_loop(0, w, fill_ones, 0)
        plsc.subcore_barrier()

        def window_body(i, _):
            wi = i * _NSUB + s

            @pl.when(wi < nw)
            def _():
                base = wi * w
                pltpu.sync_copy(idst.at[pl.ds(base, w)], dst_v)
                if gather:
                    pltpu.sync_copy(isrc.at[pl.ds(base, w)], src_v)
                    pltpu.sync_copy(table.at[c].at[src_v], rows_v)
                else:
                    pltpu.sync_copy(table.at[c].at[pl.ds(base, w)], rows_v)
                pltpu.sync_copy(rows_v, acc.at[dst_v], add=True)
                if counts:
                    @pl.when((wi % 2) == c)
                    def _():
                        pltpu.sync_copy(ones_v, cacc.at[dst_v], add=True)
            return 0
        lax.fori_loop(0, nwmax, window_body, 0)
        plsc.subcore_barrier()

        def wb_body(i, _):
            j = i * _NSUB + s

            @pl.when(j < nchunks)
            def _():
                pltpu.sync_copy(acc.at[pl.ds(j * rc, rc)],
                                out.at[c].at[pl.ds(j * rc, rc)])
                if counts:
                    pltpu.sync_copy(cacc.at[pl.ds(j * rc, rc)],
                                    ocnt.at[c].at[pl.ds(j * rc, rc)])
            return 0
        lax.fori_loop(0, ncmax, wb_body, 0)

    return pl.kernel(body, out_type=out_type, mesh=mesh,
                     scratch_types=scratch,
                     compiler_params=pltpu.CompilerParams(
                         use_tc_tiling_on_sc=False))


def _make_sc_edge_pipe(n_in, n_out, n_idx, w):
    nw = n_idx // w
    assert nw * w == n_idx and w % 8 == 0 and nw >= _NSUB
    trips = -(-nw // _NSUB)
    pairs = -(-trips // 2)
    rc = _rc_for(n_out, cap=min(w, 256))
    nchunks = n_out // rc
    ncmax = -(-nchunks // _NSUB)
    mesh = plsc.VectorSubcoreMesh(core_axis_name="c", subcore_axis_name="s")

    scratch = [
        pltpu.VMEM((w,), jnp.int32), pltpu.VMEM((w,), jnp.int32),
        pltpu.VMEM((w, 64), jnp.float32),
        pltpu.VMEM((w,), jnp.int32), pltpu.VMEM((w,), jnp.int32),
        pltpu.VMEM((w, 64), jnp.float32),
        pltpu.VMEM_SHARED((n_out, 64), jnp.float32),
        pltpu.SemaphoreType.DMA, pltpu.SemaphoreType.DMA,
        pltpu.SemaphoreType.DMA, pltpu.SemaphoreType.DMA,
    ]

    def body(table, eidx, out, src0, dst0, rows0, src1, dst1, rows1,
             acc, sem0, sem1, isem0, isem1):
        c = lax.axis_index("c")
        s = lax.axis_index("s")
        isrc = eidx.at[0]
        idst = eidx.at[1]

        _zero_rows(rows0, 64, rc)

        def zero_body(i, _):
            j = i * _NSUB + s

            @pl.when(j < nchunks)
            def _():
                pltpu.sync_copy(rows0.at[pl.ds(0, rc)],
                                acc.at[pl.ds(j * rc, rc)])
            return 0
        lax.fori_loop(0, ncmax, zero_body, 0)
        plsc.subcore_barrier()

        def wi_of(j):
            return j * _NSUB + s

        def idx_start(j, srcb, dstb, isem):
            base = wi_of(j) * w
            pltpu.async_copy(idst.at[pl.ds(base, w)], dstb, isem)
            pltpu.async_copy(isrc.at[pl.ds(base, w)], srcb, isem)

        def gather_start(j, srcb, dstb, rowsb, sem, isem):
            base = wi_of(j) * w
            pltpu.make_async_copy(idst.at[pl.ds(base, w)], dstb, isem).wait()
            pltpu.make_async_copy(isrc.at[pl.ds(base, w)], srcb, isem).wait()
            pltpu.async_copy(table.at[c].at[srcb], rowsb, sem)

        def wait_and_scatter(srcb, dstb, rowsb, sem):
            pltpu.make_async_copy(table.at[c].at[srcb], rowsb, sem).wait()
            pltpu.sync_copy(rowsb, acc.at[dstb], add=True)

        @pl.when(wi_of(0) < nw)
        def _():
            idx_start(0, src0, dst0, isem0)
            gather_start(0, src0, dst0, rows0, sem0, isem0)

        @pl.when(wi_of(1) < nw)
        def _():
            idx_start(1, src1, dst1, isem1)

        def pair_body(i, _):
            j0, j1, j2, j3 = 2 * i, 2 * i + 1, 2 * i + 2, 2 * i + 3

            @pl.when(wi_of(j1) < nw)
            def _():
                gather_start(j1, src1, dst1, rows1, sem1, isem1)

            @pl.when(wi_of(j0) < nw)
            def _():
                wait_and_scatter(src0, dst0, rows0, sem0)

            @pl.when(wi_of(j2) < nw)
            def _():
                idx_start(j2, src0, dst0, isem0)
                gather_start(j2, src0, dst0, rows0, sem0, isem0)

            @pl.when(wi_of(j1) < nw)
            def _():
                wait_and_scatter(src1, dst1, rows1, sem1)

            @pl.when(wi_of(j3) < nw)
            def _():
                idx_start(j3, src1, dst1, isem1)
            return 0
        lax.fori_loop(0, pairs, pair_body, 0)
        plsc.subcore_barrier()

        def wb_body(i, _):
            j = i * _NSUB + s

            @pl.when(j < nchunks)
            def _():
                pltpu.sync_copy(acc.at[pl.ds(j * rc, rc)],
                                out.at[c].at[pl.ds(j * rc, rc)])
            return 0
        lax.fori_loop(0, ncmax, wb_body, 0)

    return pl.kernel(body,
                     out_type=[jax.ShapeDtypeStruct((2, n_out, 64),
                                                    jnp.float32)],
                     mesh=mesh, scratch_types=scratch,
                     compiler_params=pltpu.CompilerParams(
                         use_tc_tiling_on_sc=False))


def _make_sc_edge_full(n, n_idx, w):
    nw = n_idx // w
    assert nw * w == n_idx and w % 8 == 0
    trips = -(-nw // (2 * _NSUB))
    pairs = -(-trips // 2)
    rc = _rc_for(n, cap=min(w, 256))
    nchunks = n // rc
    ncmax = -(-nchunks // _NSUB)
    mesh = plsc.VectorSubcoreMesh(core_axis_name="c", subcore_axis_name="s")

    scratch = [
        pltpu.VMEM((w,), jnp.int32), pltpu.VMEM((w,), jnp.int32),
        pltpu.VMEM((w, 128), jnp.float32),
        pltpu.VMEM((w,), jnp.int32), pltpu.VMEM((w,), jnp.int32),
        pltpu.VMEM((w, 128), jnp.float32),
        pltpu.VMEM_SHARED((n, 128), jnp.float32),
        pltpu.SemaphoreType.DMA, pltpu.SemaphoreType.DMA,
        pltpu.SemaphoreType.DMA, pltpu.SemaphoreType.DMA,
    ]

    def body(table, isrc, idst, out, src0, dst0, rows0, src1, dst1, rows1,
             acc, sem0, sem1, isem0, isem1):
        c = lax.axis_index("c")
        s = lax.axis_index("s")

        _zero_rows(rows0, 128, rc)

        def zero_body(i, _):
            j = i * _NSUB + s

            @pl.when(j < nchunks)
            def _():
                pltpu.sync_copy(rows0.at[pl.ds(0, rc)],
                                acc.at[pl.ds(j * rc, rc)])
            return 0
        lax.fori_loop(0, ncmax, zero_body, 0)
        plsc.subcore_barrier()

        def wi_of(j):
            return (j * _NSUB + s) * 2 + c

        def idx_start(j, srcb, dstb, isem):
            base = wi_of(j) * w
            pltpu.async_copy(idst.at[pl.ds(base, w)], dstb, isem)
            pltpu.async_copy(isrc.at[pl.ds(base, w)], srcb, isem)

        def gather_start(j, srcb, dstb, rowsb, sem, isem):
            base = wi_of(j) * w
            pltpu.make_async_copy(idst.at[pl.ds(base, w)], dstb, isem).wait()
            pltpu.make_async_copy(isrc.at[pl.ds(base, w)], srcb, isem).wait()
            pltpu.async_copy(table.at[srcb], rowsb, sem)

        def wait_and_scatter(srcb, dstb, rowsb, sem):
            pltpu.make_async_copy(table.at[srcb], rowsb, sem).wait()
            pltpu.sync_copy(rowsb, acc.at[dstb], add=True)

        @pl.when(wi_of(0) < nw)
        def _():
            idx_start(0, src0, dst0, isem0)
            gather_start(0, src0, dst0, rows0, sem0, isem0)

        @pl.when(wi_of(1) < nw)
        def _():
            idx_start(1, src1, dst1, isem1)

        def pair_body(i, _):
            j0, j1, j2, j3 = 2 * i, 2 * i + 1, 2 * i + 2, 2 * i + 3

            @pl.when(wi_of(j1) < nw)
            def _():
                gather_start(j1, src1, dst1, rows1, sem1, isem1)

            @pl.when(wi_of(j0) < nw)
            def _():
                wait_and_scatter(src0, dst0, rows0, sem0)

            @pl.when(wi_of(j2) < nw)
            def _():
                idx_start(j2, src0, dst0, isem0)
                gather_start(j2, src0, dst0, rows0, sem0, isem0)

            @pl.when(wi_of(j1) < nw)
            def _():
                wait_and_scatter(src1, dst1, rows1, sem1)

            @pl.when(wi_of(j3) < nw)
            def _():
                idx_start(j3, src1, dst1, isem1)
            return 0
        lax.fori_loop(0, pairs, pair_body, 0)
        plsc.subcore_barrier()

        def wb_body(i, _):
            j = i * _NSUB + s

            @pl.when(j < nchunks)
            def _():
                pltpu.sync_copy(acc.at[pl.ds(j * rc, rc)],
                                out.at[c].at[pl.ds(j * rc, rc)])
            return 0
        lax.fori_loop(0, ncmax, wb_body, 0)

    return pl.kernel(body,
                     out_type=[jax.ShapeDtypeStruct((2, n, 128),
                                                    jnp.float32)],
                     mesh=mesh, scratch_types=scratch,
                     compiler_params=pltpu.CompilerParams(
                         use_tc_tiling_on_sc=True))


_sc_edge1 = _make_sc_edge_full(_N, _E, w=160)
_sc_edge2 = _make_sc_edge_pipe(_N2, _N2, _E2, w=320)
_sc_assign = _make_sc_segsum(_N, _N2, _A, gather=True, counts=True, w=200)
_sc_pool1 = _make_sc_segsum(_N, _B, _N, gather=False, counts=True, w=400)
_sc_pool2 = _make_sc_segsum(_N2, _B, _N2, gather=False, counts=True, w=400)


def kernel(x, iso_type_2, Wi_r, Wi_n, bi, W00_r, W00_n, b00, W01_r, W01_n, b01,
           W10_r, W10_n, b10, W11_r, W11_n, b11,
           fc0_w, fc0_b, fc1_w, fc1_b, fc2_w, fc2_b,
           edge_index, edge_index_2, assignment_index_2, batch, batch_2):
    src, dst = edge_index[0], edge_index[1]
    asrc, adst = assignment_index_2[0], assignment_index_2[1]

    r0, t0 = _tc_init(x, Wi_r, Wi_n)
    (s0,) = _sc_edge1(t0, src, dst)
    r1, t1 = _tc_step1(r0, s0, bi, W00_r, W00_n)
    (s1,) = _sc_edge1(t1, src, dst)
    r2, t2 = _tc_step1(r1, s1, b00, W01_r, W01_n)
    (s2,) = _sc_edge1(t2, src, dst)
    hs = _tc_fin1(r2, s2, b01)

    p1s, p1c = _sc_pool1(hs, batch, batch)

    asum, acnt = _sc_assign(hs, asrc, adst)
    r3, t3 = _tc_assign(asum, acnt, iso_type_2, W10_r, W10_n)
    (s3,) = _sc_edge2(t3, edge_index_2)
    r4, t4 = _tc_step(r3, s3, b10, W11_r, W11_n)
    (s4,) = _sc_edge2(t4, edge_index_2)
    h2s = _tc_fin(r4, s4, b11)

    p2s, p2c = _sc_pool2(h2s, batch_2, batch_2)

    return _tc_head(p1s, p1c, p2s, p2c, fc0_w, fc0_b, fc1_w, fc1_b,
                    fc2_w, fc2_b)


# speedup vs baseline: 9.1101x; 1.0311x over previous
import functools
import math

import jax
import jax.numpy as jnp
from jax import lax
from jax.experimental import pallas as pl
from jax.experimental.pallas import tpu as pltpu
from jax.experimental.pallas import tpu_sc as plsc

_N, _E, _N2, _E2, _A, _B = 10000, 320000, 20000, 320000, 40000, 64
_D, _ISO, _NC = 128, 16, 10
_NSC = 2
_NSUB = 16
_W = 1000


def _elu(x):
    return jnp.where(x > 0, x, jnp.exp(jnp.minimum(x, 0.0)) - 1.0)


_BR = 1000


def _dot(a, b):
    return jnp.dot(a, b, preferred_element_type=jnp.float32)


def _k_step1_body(h_ref, s_ref, b_ref, wr_ref, wn_ref, o_ref):
    o_ref[...] = _elu(_dot(h_ref[...], wr_ref[...])
                      + _dot(s_ref[0] + s_ref[1], wn_ref[...])
                      + b_ref[...])


def _k_fin1_body(h_ref, s_ref, b_ref, wr_ref, wn_ref, o_ref):
    h = _elu(_dot(h_ref[...], wr_ref[...])
             + _dot(s_ref[0] + s_ref[1], wn_ref[...])
             + b_ref[...])
    o_ref[0] = h[:, :64]
    o_ref[1] = h[:, 64:]


def _k_step_body(r_ref, s_ref, b_ref, wa_ref, wb_ref, r2_ref, t2_ref):
    s = jnp.concatenate([s_ref[0], s_ref[1]], axis=1)
    h = _elu(r_ref[...] + s + b_ref[...])
    r2_ref[...] = _dot(h, wa_ref[...])
    t = _dot(h, wb_ref[...])
    t2_ref[0] = t[:, :64]
    t2_ref[1] = t[:, 64:]


def _k_fin_body(r_ref, s_ref, b_ref, o_ref):
    s = jnp.concatenate([s_ref[0], s_ref[1]], axis=1)
    h = _elu(r_ref[...] + s + b_ref[...])
    o_ref[0] = h[:, :64]
    o_ref[1] = h[:, 64:]


def _k_assign_body(ss_ref, cc_ref, iso_ref, wr_ref, wn_ref, r_ref, t_ref):
    cnt = jnp.maximum(cc_ref[0][:, :1] + cc_ref[1][:, :1], 1.0)
    m = jnp.concatenate([ss_ref[0], ss_ref[1]], axis=1) / cnt
    iso = iso_ref[...]
    r_ref[...] = _dot(m, wr_ref[:128]) + _dot(iso, wr_ref[128:])
    t = _dot(m, wn_ref[:128]) + _dot(iso, wn_ref[128:])
    t_ref[0] = t[:, :64]
    t_ref[1] = t[:, 64:]


def _k_head_body(p1s_ref, p1c_ref, p2s_ref, p2c_ref,
                 w0_ref, b0_ref, w1_ref, b1_ref, w2_ref, b2_ref, o_ref):
    c1 = jnp.maximum(p1c_ref[0][:, :1] + p1c_ref[1][:, :1], 1.0)
    p1 = jnp.concatenate([p1s_ref[0], p1s_ref[1]], axis=1) / c1
    c2 = jnp.maximum(p2c_ref[0][:, :1] + p2c_ref[1][:, :1], 1.0)
    p2 = jnp.concatenate([p2s_ref[0], p2s_ref[1]], axis=1) / c2
    g = jnp.concatenate([p1, p2], axis=1)
    g = _elu(_dot(g, w0_ref[...]) + b0_ref[...])
    g = _elu(_dot(g, w1_ref[...]) + b1_ref[...])
    g = _dot(g, w2_ref[...]) + b2_ref[...]
    m = jnp.max(g, axis=1, keepdims=True)
    z = g - m
    o_ref[...] = z - jnp.log(jnp.sum(jnp.exp(z), axis=1, keepdims=True))


def _row_spec(n):
    return pl.BlockSpec((_BR, n), lambda i: (i, 0))


def _split_spec():
    return pl.BlockSpec((2, _BR, 64), lambda i: (0, i, 0))


def _full_spec(shape):
    return pl.BlockSpec(shape, lambda i: tuple(0 for _ in shape))


def _part_spec():
    return pl.BlockSpec((2, _BR, 128), lambda i: (0, i, 0))


def _tc_step1(h, s, b, wr, wn):
    n = h.shape[0]
    return pl.pallas_call(
        _k_step1_body,
        grid=(n // _BR,),
        in_specs=[_row_spec(128), _part_spec(), _full_spec((1, 128)),
                  _full_spec((128, 128)), _full_spec((128, 128))],
        out_specs=_row_spec(128),
        out_shape=jax.ShapeDtypeStruct((n, 128), jnp.float32),
    )(h, s, b.reshape(1, 128), wr, wn)


def _tc_fin1(h, s, b, wr, wn):
    n = h.shape[0]
    return pl.pallas_call(
        _k_fin1_body,
        grid=(n // _BR,),
        in_specs=[_row_spec(128), _part_spec(), _full_spec((1, 128)),
                  _full_spec((128, 128)), _full_spec((128, 128))],
        out_specs=_split_spec(),
        out_shape=jax.ShapeDtypeStruct((2, n, 64), jnp.float32),
    )(h, s, b.reshape(1, 128), wr, wn)


def _tc_step(r, s, b, wa, wb):
    n = r.shape[0]
    return pl.pallas_call(
        _k_step_body,
        grid=(n // _BR,),
        in_specs=[_row_spec(128), _split_spec(), _full_spec((1, 128)),
                  _full_spec((128, 128)), _full_spec((128, 128))],
        out_specs=[_row_spec(128), _split_spec()],
        out_shape=[jax.ShapeDtypeStruct((n, 128), jnp.float32),
                   jax.ShapeDtypeStruct((2, n, 64), jnp.float32)],
    )(r, s, b.reshape(1, 128), wa, wb)


def _tc_fin(r, s, b):
    n = r.shape[0]
    return pl.pallas_call(
        _k_fin_body,
        grid=(n // _BR,),
        in_specs=[_row_spec(128), _split_spec(), _full_spec((1, 128))],
        out_specs=_split_spec(),
        out_shape=jax.ShapeDtypeStruct((2, n, 64), jnp.float32),
    )(r, s, b.reshape(1, 128))


def _tc_assign(ss, cc, iso, wr, wn):
    n = ss.shape[1]
    return pl.pallas_call(
        _k_assign_body,
        grid=(n // _BR,),
        in_specs=[_split_spec(),
                  pl.BlockSpec((2, _BR, 16), lambda i: (0, i, 0)),
                  _row_spec(16),
                  _full_spec((144, 128)), _full_spec((144, 128))],
        out_specs=[_row_spec(128), _split_spec()],
        out_shape=[jax.ShapeDtypeStruct((n, 128), jnp.float32),
                   jax.ShapeDtypeStruct((2, n, 64), jnp.float32)],
    )(ss, cc, iso, wr, wn)


def _tc_head(p1s, p1c, p2s, p2c, w0, b0, w1, b1, w2, b2):
    return pl.pallas_call(
        _k_head_body,
        grid=(1,),
        in_specs=[_full_spec((2, 64, 64)), _full_spec((2, 64, 16)),
                  _full_spec((2, 64, 64)), _full_spec((2, 64, 16)),
                  _full_spec((256, 128)), _full_spec((1, 128)),
                  _full_spec((128, 64)), _full_spec((1, 64)),
                  _full_spec((64, 10)), _full_spec((1, 10))],
        out_specs=_full_spec((64, 10)),
        out_shape=jax.ShapeDtypeStruct((64, 10), jnp.float32),
    )(p1s, p1c, p2s, p2c, w0, b0.reshape(1, 128), w1, b1.reshape(1, 64),
      w2, b2.reshape(1, 10))


def _zero_rows(zbuf, width, nrows):
    zv = jnp.zeros((16,), jnp.float32)

    def fill(i, _):
        for j in range(width // 16):
            zbuf[i, pl.ds(j * 16, 16)] = zv
        return 0
    lax.fori_loop(0, nrows, fill, 0)


def _rc_for(n_out, cap=256):
    best = 8
    for cand in range(8, cap + 1, 8):
        if n_out % cand == 0:
            best = cand
    return best


def _make_sc_segsum(n_in, n_out, n_idx, *, gather, counts, w):
    nw = n_idx // w
    assert nw * w == n_idx and w % 8 == 0
    nwmax = -(-nw // _NSUB)
    rc = _rc_for(n_out, cap=min(w, 256))
    nchunks = n_out // rc
    ncmax = -(-nchunks // _NSUB)
    mesh = plsc.VectorSubcoreMesh(core_axis_name="c", subcore_axis_name="s")

    out_type = [jax.ShapeDtypeStruct((2, n_out, 64), jnp.float32)]
    scratch = [
        pltpu.VMEM((w,), jnp.int32),
        pltpu.VMEM((w,), jnp.int32),
        pltpu.VMEM((w, 64), jnp.float32),
        pltpu.VMEM_SHARED((n_out, 64), jnp.float32),
    ]
    if counts:
        out_type.append(jax.ShapeDtypeStruct((2, n_out, 16), jnp.float32))
        scratch += [
            pltpu.VMEM((w, 16), jnp.float32),
            pltpu.VMEM_SHARED((n_out, 16), jnp.float32),
        ]

    def body(table, isrc, idst, *refs):
        if counts:
            out, ocnt, src_v, dst_v, rows_v, acc, ones_v, cacc = refs
        else:
            out, src_v, dst_v, rows_v, acc = refs
            ocnt = ones_v = cacc = None
        c = lax.axis_index("c")
        s = lax.axis_index("s")

        _zero_rows(rows_v, 64, rc)
        if counts:
            _zero_rows(ones_v, 16, rc)

        def zero_body(i, _):
            j = i * _NSUB + s

            @pl.when(j < nchunks)
            def _():
                pltpu.sync_copy(rows_v.at[pl.ds(0, rc)],
                                acc.at[pl.ds(j * rc, rc)])
                if counts:
                    pltpu.sync_copy(ones_v.at[pl.ds(0, rc)],
                                    cacc.at[pl.ds(j * rc, rc)])
            return 0
        lax.fori_loop(0, ncmax, zero_body, 0)

        if counts:
            ov = jnp.ones((16,), jnp.float32)

            def fill_ones(i, _):
                ones_v[i, pl.ds(0, 16)] = ov
                return 0
            lax.fori_loop(0, w, fill_ones, 0)
        plsc.subcore_barrier()

        def window_body(i, _):
            wi = i * _NSUB + s

            @pl.when(wi < nw)
            def _():
                base = wi * w
                pltpu.sync_copy(idst.at[pl.ds(base, w)], dst_v)
                if gather:
                    pltpu.sync_copy(isrc.at[pl.ds(base, w)], src_v)
                    pltpu.sync_copy(table.at[c].at[src_v], rows_v)
                else:
                    pltpu.sync_copy(table.at[c].at[pl.ds(base, w)], rows_v)
                pltpu.sync_copy(rows_v, acc.at[dst_v], add=True)
                if counts:
                    @pl.when((wi % 2) == c)
                    def _():
                        pltpu.sync_copy(ones_v, cacc.at[dst_v], add=True)
            return 0
        lax.fori_loop(0, nwmax, window_body, 0)
        plsc.subcore_barrier()

        def wb_body(i, _):
            j = i * _NSUB + s

            @pl.when(j < nchunks)
            def _():
                pltpu.sync_copy(acc.at[pl.ds(j * rc, rc)],
                                out.at[c].at[pl.ds(j * rc, rc)])
                if counts:
                    pltpu.sync_copy(cacc.at[pl.ds(j * rc, rc)],
                                    ocnt.at[c].at[pl.ds(j * rc, rc)])
            return 0
        lax.fori_loop(0, ncmax, wb_body, 0)

    return pl.kernel(body, out_type=out_type, mesh=mesh,
                     scratch_types=scratch,
                     compiler_params=pltpu.CompilerParams(
                         use_tc_tiling_on_sc=False))


def _make_sc_edge_pipe(n_in, n_out, n_idx, w):
    nw = n_idx // w
    assert nw * w == n_idx and w % 8 == 0 and nw >= _NSUB
    trips = -(-nw // _NSUB)
    pairs = -(-trips // 2)
    rc = _rc_for(n_out, cap=min(w, 256))
    nchunks = n_out // rc
    ncmax = -(-nchunks // _NSUB)
    mesh = plsc.VectorSubcoreMesh(core_axis_name="c", subcore_axis_name="s")

    scratch = [
        pltpu.VMEM((w,), jnp.int32), pltpu.VMEM((w,), jnp.int32),
        pltpu.VMEM((w, 64), jnp.float32),
        pltpu.VMEM((w,), jnp.int32), pltpu.VMEM((w,), jnp.int32),
        pltpu.VMEM((w, 64), jnp.float32),
        pltpu.VMEM_SHARED((n_out, 64), jnp.float32),
        pltpu.SemaphoreType.DMA, pltpu.SemaphoreType.DMA,
        pltpu.SemaphoreType.DMA, pltpu.SemaphoreType.DMA,
    ]

    def body(table, eidx, out, src0, dst0, rows0, src1, dst1, rows1,
             acc, sem0, sem1, isem0, isem1):
        c = lax.axis_index("c")
        s = lax.axis_index("s")
        isrc = eidx.at[0]
        idst = eidx.at[1]

        _zero_rows(rows0, 64, rc)

        def zero_body(i, _):
            j = i * _NSUB + s

            @pl.when(j < nchunks)
            def _():
                pltpu.sync_copy(rows0.at[pl.ds(0, rc)],
                                acc.at[pl.ds(j * rc, rc)])
            return 0
        lax.fori_loop(0, ncmax, zero_body, 0)
        plsc.subcore_barrier()

        def wi_of(j):
            return j * _NSUB + s

        def idx_start(j, srcb, dstb, isem):
            base = wi_of(j) * w
            pltpu.async_copy(idst.at[pl.ds(base, w)], dstb, isem)
            pltpu.async_copy(isrc.at[pl.ds(base, w)], srcb, isem)

        def gather_start(j, srcb, dstb, rowsb, sem, isem):
            base = wi_of(j) * w
            pltpu.make_async_copy(idst.at[pl.ds(base, w)], dstb, isem).wait()
            pltpu.make_async_copy(isrc.at[pl.ds(base, w)], srcb, isem).wait()
            pltpu.async_copy(table.at[c].at[srcb], rowsb, sem)

        def wait_and_scatter(srcb, dstb, rowsb, sem):
            pltpu.make_async_copy(table.at[c].at[srcb], rowsb, sem).wait()
            pltpu.sync_copy(rowsb, acc.at[dstb], add=True)

        @pl.when(wi_of(0) < nw)
        def _():
            idx_start(0, src0, dst0, isem0)
            gather_start(0, src0, dst0, rows0, sem0, isem0)

        @pl.when(wi_of(1) < nw)
        def _():
            idx_start(1, src1, dst1, isem1)

        def pair_body(i, _):
            j0, j1, j2, j3 = 2 * i, 2 * i + 1, 2 * i + 2, 2 * i + 3

            @pl.when(wi_of(j1) < nw)
            def _():
                gather_start(j1, src1, dst1, rows1, sem1, isem1)

            @pl.when(wi_of(j0) < nw)
            def _():
                wait_and_scatter(src0, dst0, rows0, sem0)

            @pl.when(wi_of(j2) < nw)
            def _():
                idx_start(j2, src0, dst0, isem0)
                gather_start(j2, src0, dst0, rows0, sem0, isem0)

            @pl.when(wi_of(j1) < nw)
            def _():
                wait_and_scatter(src1, dst1, rows1, sem1)

            @pl.when(wi_of(j3) < nw)
            def _():
                idx_start(j3, src1, dst1, isem1)
            return 0
        lax.fori_loop(0, pairs, pair_body, 0)
        plsc.subcore_barrier()

        def wb_body(i, _):
            j = i * _NSUB + s

            @pl.when(j < nchunks)
            def _():
                pltpu.sync_copy(acc.at[pl.ds(j * rc, rc)],
                                out.at[c].at[pl.ds(j * rc, rc)])
            return 0
        lax.fori_loop(0, ncmax, wb_body, 0)

    return pl.kernel(body,
                     out_type=[jax.ShapeDtypeStruct((2, n_out, 64),
                                                    jnp.float32)],
                     mesh=mesh, scratch_types=scratch,
                     compiler_params=pltpu.CompilerParams(
                         use_tc_tiling_on_sc=False))


def _make_sc_edge_full(n, n_idx, w):
    nw = n_idx // w
    assert nw * w == n_idx and w % 8 == 0
    trips = -(-nw // (2 * _NSUB))
    pairs = -(-trips // 2)
    rc = _rc_for(n, cap=min(w, 256))
    nchunks = n // rc
    ncmax = -(-nchunks // _NSUB)
    mesh = plsc.VectorSubcoreMesh(core_axis_name="c", subcore_axis_name="s")

    scratch = [
        pltpu.VMEM((w,), jnp.int32), pltpu.VMEM((w,), jnp.int32),
        pltpu.VMEM((w, 128), jnp.float32),
        pltpu.VMEM((w,), jnp.int32), pltpu.VMEM((w,), jnp.int32),
        pltpu.VMEM((w, 128), jnp.float32),
        pltpu.VMEM_SHARED((n, 128), jnp.float32),
        pltpu.SemaphoreType.DMA, pltpu.SemaphoreType.DMA,
        pltpu.SemaphoreType.DMA, pltpu.SemaphoreType.DMA,
    ]

    def body(table, isrc, idst, out, src0, dst0, rows0, src1, dst1, rows1,
             acc, sem0, sem1, isem0, isem1):
        c = lax.axis_index("c")
        s = lax.axis_index("s")

        _zero_rows(rows0, 128, rc)

        def zero_body(i, _):
            j = i * _NSUB + s

            @pl.when(j < nchunks)
            def _():
                pltpu.sync_copy(rows0.at[pl.ds(0, rc)],
                                acc.at[pl.ds(j * rc, rc)])
            return 0
        lax.fori_loop(0, ncmax, zero_body, 0)
        plsc.subcore_barrier()

        def wi_of(j):
            return (j * _NSUB + s) * 2 + c

        def idx_start(j, srcb, dstb, isem):
            base = wi_of(j) * w
            pltpu.async_copy(idst.at[pl.ds(base, w)], dstb, isem)
            pltpu.async_copy(isrc.at[pl.ds(base, w)], srcb, isem)

        def gather_start(j, srcb, dstb, rowsb, sem, isem):
            base = wi_of(j) * w
            pltpu.make_async_copy(idst.at[pl.ds(base, w)], dstb, isem).wait()
            pltpu.make_async_copy(isrc.at[pl.ds(base, w)], srcb, isem).wait()
            pltpu.async_copy(table.at[srcb], rowsb, sem)

        def wait_and_scatter(srcb, dstb, rowsb, sem):
            pltpu.make_async_copy(table.at[srcb], rowsb, sem).wait()
            pltpu.sync_copy(rowsb, acc.at[dstb], add=True)

        @pl.when(wi_of(0) < nw)
        def _():
            idx_start(0, src0, dst0, isem0)
            gather_start(0, src0, dst0, rows0, sem0, isem0)

        @pl.when(wi_of(1) < nw)
        def _():
            idx_start(1, src1, dst1, isem1)

        def pair_body(i, _):
            j0, j1, j2, j3 = 2 * i, 2 * i + 1, 2 * i + 2, 2 * i + 3

            @pl.when(wi_of(j1) < nw)
            def _():
                gather_start(j1, src1, dst1, rows1, sem1, isem1)

            @pl.when(wi_of(j0) < nw)
            def _():
                wait_and_scatter(src0, dst0, rows0, sem0)

            @pl.when(wi_of(j2) < nw)
            def _():
                idx_start(j2, src0, dst0, isem0)
                gather_start(j2, src0, dst0, rows0, sem0, isem0)

            @pl.when(wi_of(j1) < nw)
            def _():
                wait_and_scatter(src1, dst1, rows1, sem1)

            @pl.when(wi_of(j3) < nw)
            def _():
                idx_start(j3, src1, dst1, isem1)
            return 0
        lax.fori_loop(0, pairs, pair_body, 0)
        plsc.subcore_barrier()

        def wb_body(i, _):
            j = i * _NSUB + s

            @pl.when(j < nchunks)
            def _():
                pltpu.sync_copy(acc.at[pl.ds(j * rc, rc)],
                                out.at[c].at[pl.ds(j * rc, rc)])
            return 0
        lax.fori_loop(0, ncmax, wb_body, 0)

    return pl.kernel(body,
                     out_type=[jax.ShapeDtypeStruct((2, n, 128),
                                                    jnp.float32)],
                     mesh=mesh, scratch_types=scratch,
                     compiler_params=pltpu.CompilerParams(
                         use_tc_tiling_on_sc=True))


def _make_sc_assign_pipe(n_in, n_out, n_idx, w):
    nw = n_idx // w
    assert nw * w == n_idx and w % 8 == 0 and nw >= _NSUB
    trips = -(-nw // _NSUB)
    pairs = -(-trips // 2)
    rc = _rc_for(n_out, cap=min(w, 256))
    nchunks = n_out // rc
    ncmax = -(-nchunks // _NSUB)
    mesh = plsc.VectorSubcoreMesh(core_axis_name="c", subcore_axis_name="s")

    scratch = [
        pltpu.VMEM((w,), jnp.int32), pltpu.VMEM((w,), jnp.int32),
        pltpu.VMEM((w, 64), jnp.float32),
        pltpu.VMEM((w,), jnp.int32), pltpu.VMEM((w,), jnp.int32),
        pltpu.VMEM((w, 64), jnp.float32),
        pltpu.VMEM((w, 16), jnp.float32),
        pltpu.VMEM_SHARED((n_out, 64), jnp.float32),
        pltpu.VMEM_SHARED((n_out, 16), jnp.float32),
        pltpu.SemaphoreType.DMA, pltpu.SemaphoreType.DMA,
        pltpu.SemaphoreType.DMA, pltpu.SemaphoreType.DMA,
    ]
    out_type = [jax.ShapeDtypeStruct((2, n_out, 64), jnp.float32),
                jax.ShapeDtypeStruct((2, n_out, 16), jnp.float32)]

    def body(table, isrc, idst, out, ocnt, src0, dst0, rows0, src1, dst1,
             rows1, ones_v, acc, cacc, sem0, sem1, isem0, isem1):
        c = lax.axis_index("c")
        s = lax.axis_index("s")

        _zero_rows(rows0, 64, rc)
        _zero_rows(ones_v, 16, rc)

        def zero_body(i, _):
            j = i * _NSUB + s

            @pl.when(j < nchunks)
            def _():
                pltpu.sync_copy(rows0.at[pl.ds(0, rc)],
                                acc.at[pl.ds(j * rc, rc)])
                pltpu.sync_copy(ones_v.at[pl.ds(0, rc)],
                                cacc.at[pl.ds(j * rc, rc)])
            return 0
        lax.fori_loop(0, ncmax, zero_body, 0)

        ov = jnp.ones((16,), jnp.float32)

        def fill_ones(i, _):
            ones_v[i, pl.ds(0, 16)] = ov
            return 0
        lax.fori_loop(0, w, fill_ones, 0)
        plsc.subcore_barrier()

        def wi_of(j):
            return j * _NSUB + s

        def idx_start(j, srcb, dstb, isem):
            base = wi_of(j) * w
            pltpu.async_copy(idst.at[pl.ds(base, w)], dstb, isem)
            pltpu.async_copy(isrc.at[pl.ds(base, w)], srcb, isem)

        def gather_start(j, srcb, dstb, rowsb, sem, isem):
            base = wi_of(j) * w
            pltpu.make_async_copy(idst.at[pl.ds(base, w)], dstb, isem).wait()
            pltpu.make_async_copy(isrc.at[pl.ds(base, w)], srcb, isem).wait()
            pltpu.async_copy(table.at[c].at[srcb], rowsb, sem)

        def wait_and_scatter(j, srcb, dstb, rowsb, sem):
            pltpu.make_async_copy(table.at[c].at[srcb], rowsb, sem).wait()
            pltpu.sync_copy(rowsb, acc.at[dstb], add=True)

            @pl.when((wi_of(j) % 2) == c)
            def _():
                pltpu.sync_copy(ones_v, cacc.at[dstb], add=True)

        @pl.when(wi_of(0) < nw)
        def _():
            idx_start(0, src0, dst0, isem0)
            gather_start(0, src0, dst0, rows0, sem0, isem0)

        @pl.when(wi_of(1) < nw)
        def _():
            idx_start(1, src1, dst1, isem1)

        def pair_body(i, _):
            j0, j1, j2, j3 = 2 * i, 2 * i + 1, 2 * i + 2, 2 * i + 3

            @pl.when(wi_of(j1) < nw)
            def _():
                gather_start(j1, src1, dst1, rows1, sem1, isem1)

            @pl.when(wi_of(j0) < nw)
            def _():
                wait_and_scatter(j0, src0, dst0, rows0, sem0)

            @pl.when(wi_of(j2) < nw)
            def _():
                idx_start(j2, src0, dst0, isem0)
                gather_start(j2, src0, dst0, rows0, sem0, isem0)

            @pl.when(wi_of(j1) < nw)
            def _():
                wait_and_scatter(j1, src1, dst1, rows1, sem1)

            @pl.when(wi_of(j3) < nw)
            def _():
                idx_start(j3, src1, dst1, isem1)
            return 0
        lax.fori_loop(0, pairs, pair_body, 0)
        plsc.subcore_barrier()

        def wb_body(i, _):
            j = i * _NSUB + s

            @pl.when(j < nchunks)
            def _():
                pltpu.sync_copy(acc.at[pl.ds(j * rc, rc)],
                                out.at[c].at[pl.ds(j * rc, rc)])
                pltpu.sync_copy(cacc.at[pl.ds(j * rc, rc)],
                                ocnt.at[c].at[pl.ds(j * rc, rc)])
            return 0
        lax.fori_loop(0, ncmax, wb_body, 0)

    return pl.kernel(body, out_type=out_type, mesh=mesh,
                     scratch_types=scratch,
                     compiler_params=pltpu.CompilerParams(
                         use_tc_tiling_on_sc=False))


_sc_edge1 = _make_sc_edge_full(_N, _E, w=160)
_sc_edge2 = _make_sc_edge_pipe(_N2, _N2, _E2, w=320)
_sc_assign = _make_sc_assign_pipe(_N, _N2, _A, w=160)
_sc_pool1 = _make_sc_segsum(_N, _B, _N, gather=False, counts=True, w=400)
_sc_pool2 = _make_sc_segsum(_N2, _B, _N2, gather=False, counts=True, w=400)


def kernel(x, iso_type_2, Wi_r, Wi_n, bi, W00_r, W00_n, b00, W01_r, W01_n, b01,
           W10_r, W10_n, b10, W11_r, W11_n, b11,
           fc0_w, fc0_b, fc1_w, fc1_b, fc2_w, fc2_b,
           edge_index, edge_index_2, assignment_index_2, batch, batch_2):
    src, dst = edge_index[0], edge_index[1]
    asrc, adst = assignment_index_2[0], assignment_index_2[1]

    (s0,) = _sc_edge1(x, src, dst)
    h1 = _tc_step1(x, s0, bi, Wi_r, Wi_n)
    (s1,) = _sc_edge1(h1, src, dst)
    h2 = _tc_step1(h1, s1, b00, W00_r, W00_n)
    (s2,) = _sc_edge1(h2, src, dst)
    hs = _tc_fin1(h2, s2, b01, W01_r, W01_n)

    p1s, p1c = _sc_pool1(hs, batch, batch)

    asum, acnt = _sc_assign(hs, asrc, adst)
    r3, t3 = _tc_assign(asum, acnt, iso_type_2, W10_r, W10_n)
    (s3,) = _sc_edge2(t3, edge_index_2)
    r4, t4 = _tc_step(r3, s3, b10, W11_r, W11_n)
    (s4,) = _sc_edge2(t4, edge_index_2)
    h2s = _tc_fin(r4, s4, b11)

    p2s, p2c = _sc_pool2(h2s, batch_2, batch_2)

    return _tc_head(p1s, p1c, p2s, p2c, fc0_w, fc0_b, fc1_w, fc1_b,
                    fc2_w, fc2_b)


# speedup vs baseline: 9.1194x; 1.0010x over previous
import jax
import jax.numpy as jnp
from jax import lax
from jax.experimental import pallas as pl
from jax.experimental.pallas import tpu as pltpu
from jax.experimental.pallas import tpu_sc as plsc

_N, _E, _N2, _E2, _A, _B = 10000, 320000, 20000, 320000, 40000, 64
_D, _ISO, _NC = 128, 16, 10
_NSUB = 16


def _elu(x):
    return jnp.where(x > 0, x, jnp.exp(jnp.minimum(x, 0.0)) - 1.0)


_BR = 1000


def _dot(a, b):
    return jnp.dot(a, b, preferred_element_type=jnp.float32)


def _k_step1_body(h_ref, s_ref, b_ref, wr_ref, wn_ref, o_ref):
    o_ref[...] = _elu(_dot(h_ref[...], wr_ref[...])
                      + _dot(s_ref[0] + s_ref[1], wn_ref[...])
                      + b_ref[...])


def _k_fin1_body(h_ref, s_ref, b_ref, wr_ref, wn_ref, o_ref):
    h = _elu(_dot(h_ref[...], wr_ref[...])
             + _dot(s_ref[0] + s_ref[1], wn_ref[...])
             + b_ref[...])
    o_ref[0] = h[:, :64]
    o_ref[1] = h[:, 64:]


def _k_step_body(r_ref, s_ref, b_ref, wa_ref, wb_ref, r2_ref, t2_ref):
    s = jnp.concatenate([s_ref[0], s_ref[1]], axis=1)
    h = _elu(r_ref[...] + s + b_ref[...])
    r2_ref[...] = _dot(h, wa_ref[...])
    t = _dot(h, wb_ref[...])
    t2_ref[0] = t[:, :64]
    t2_ref[1] = t[:, 64:]


def _k_fin_body(r_ref, s_ref, b_ref, o_ref):
    s = jnp.concatenate([s_ref[0], s_ref[1]], axis=1)
    h = _elu(r_ref[...] + s + b_ref[...])
    o_ref[0] = h[:, :64]
    o_ref[1] = h[:, 64:]


def _k_assign_body(ss_ref, cc_ref, iso_ref, wr_ref, wn_ref, r_ref, t_ref):
    cnt = jnp.maximum(cc_ref[0][:, :1] + cc_ref[1][:, :1], 1.0)
    m = jnp.concatenate([ss_ref[0], ss_ref[1]], axis=1) / cnt
    iso = iso_ref[...]
    r_ref[...] = _dot(m, wr_ref[:128]) + _dot(iso, wr_ref[128:])
    t = _dot(m, wn_ref[:128]) + _dot(iso, wn_ref[128:])
    t_ref[0] = t[:, :64]
    t_ref[1] = t[:, 64:]


def _k_head_body(p1s_ref, p1c_ref, p2s_ref, p2c_ref,
                 w0_ref, b0_ref, w1_ref, b1_ref, w2_ref, b2_ref, o_ref):
    c1 = jnp.maximum(p1c_ref[0][:, :1] + p1c_ref[1][:, :1], 1.0)
    p1 = jnp.concatenate([p1s_ref[0], p1s_ref[1]], axis=1) / c1
    c2 = jnp.maximum(p2c_ref[0][:, :1] + p2c_ref[1][:, :1], 1.0)
    p2 = jnp.concatenate([p2s_ref[0], p2s_ref[1]], axis=1) / c2
    g = jnp.concatenate([p1, p2], axis=1)
    g = _elu(_dot(g, w0_ref[...]) + b0_ref[...])
    g = _elu(_dot(g, w1_ref[...]) + b1_ref[...])
    g = _dot(g, w2_ref[...]) + b2_ref[...]
    m = jnp.max(g, axis=1, keepdims=True)
    z = g - m
    o_ref[...] = z - jnp.log(jnp.sum(jnp.exp(z), axis=1, keepdims=True))


def _row_spec(n):
    return pl.BlockSpec((_BR, n), lambda i: (i, 0))


def _split_spec():
    return pl.BlockSpec((2, _BR, 64), lambda i: (0, i, 0))


def _full_spec(shape):
    return pl.BlockSpec(shape, lambda i: tuple(0 for _ in shape))


def _part_spec():
    return pl.BlockSpec((2, _BR, 128), lambda i: (0, i, 0))


def _tc_step1(h, s, b, wr, wn):
    n = h.shape[0]
    return pl.pallas_call(
        _k_step1_body,
        grid=(n // _BR,),
        in_specs=[_row_spec(128), _part_spec(), _full_spec((1, 128)),
                  _full_spec((128, 128)), _full_spec((128, 128))],
        out_specs=_row_spec(128),
        out_shape=jax.ShapeDtypeStruct((n, 128), jnp.float32),
    )(h, s, b.reshape(1, 128), wr, wn)


def _tc_fin1(h, s, b, wr, wn):
    n = h.shape[0]
    return pl.pallas_call(
        _k_fin1_body,
        grid=(n // _BR,),
        in_specs=[_row_spec(128), _part_spec(), _full_spec((1, 128)),
                  _full_spec((128, 128)), _full_spec((128, 128))],
        out_specs=_split_spec(),
        out_shape=jax.ShapeDtypeStruct((2, n, 64), jnp.float32),
    )(h, s, b.reshape(1, 128), wr, wn)


def _tc_step(r, s, b, wa, wb):
    n = r.shape[0]
    return pl.pallas_call(
        _k_step_body,
        grid=(n // _BR,),
        in_specs=[_row_spec(128), _split_spec(), _full_spec((1, 128)),
                  _full_spec((128, 128)), _full_spec((128, 128))],
        out_specs=[_row_spec(128), _split_spec()],
        out_shape=[jax.ShapeDtypeStruct((n, 128), jnp.float32),
                   jax.ShapeDtypeStruct((2, n, 64), jnp.float32)],
    )(r, s, b.reshape(1, 128), wa, wb)


def _tc_fin(r, s, b):
    n = r.shape[0]
    return pl.pallas_call(
        _k_fin_body,
        grid=(n // _BR,),
        in_specs=[_row_spec(128), _split_spec(), _full_spec((1, 128))],
        out_specs=_split_spec(),
        out_shape=jax.ShapeDtypeStruct((2, n, 64), jnp.float32),
    )(r, s, b.reshape(1, 128))


def _tc_assign(ss, cc, iso, wr, wn):
    n = ss.shape[1]
    return pl.pallas_call(
        _k_assign_body,
        grid=(n // _BR,),
        in_specs=[_split_spec(),
                  pl.BlockSpec((2, _BR, 16), lambda i: (0, i, 0)),
                  _row_spec(16),
                  _full_spec((144, 128)), _full_spec((144, 128))],
        out_specs=[_row_spec(128), _split_spec()],
        out_shape=[jax.ShapeDtypeStruct((n, 128), jnp.float32),
                   jax.ShapeDtypeStruct((2, n, 64), jnp.float32)],
    )(ss, cc, iso, wr, wn)


def _tc_head(p1s, p1c, p2s, p2c, w0, b0, w1, b1, w2, b2):
    return pl.pallas_call(
        _k_head_body,
        grid=(1,),
        in_specs=[_full_spec((2, 64, 64)), _full_spec((2, 64, 16)),
                  _full_spec((2, 64, 64)), _full_spec((2, 64, 16)),
                  _full_spec((256, 128)), _full_spec((1, 128)),
                  _full_spec((128, 64)), _full_spec((1, 64)),
                  _full_spec((64, 10)), _full_spec((1, 10))],
        out_specs=_full_spec((64, 10)),
        out_shape=jax.ShapeDtypeStruct((64, 10), jnp.float32),
    )(p1s, p1c, p2s, p2c, w0, b0.reshape(1, 128), w1, b1.reshape(1, 64),
      w2, b2.reshape(1, 10))


def _zero_rows(zbuf, width, nrows):
    zv = jnp.zeros((16,), jnp.float32)

    def fill(i, _):
        for j in range(width // 16):
            zbuf[i, pl.ds(j * 16, 16)] = zv
        return 0
    lax.fori_loop(0, nrows, fill, 0)


def _rc_for(n_out, cap=256):
    best = 8
    for cand in range(8, cap + 1, 8):
        if n_out % cand == 0:
            best = cand
    return best


def _make_sc_segsum(n_in, n_out, n_idx, *, gather, counts, w):
    nw = n_idx // w
    assert nw * w == n_idx and w % 8 == 0
    nwmax = -(-nw // _NSUB)
    rc = _rc_for(n_out, cap=min(w, 256))
    nchunks = n_out // rc
    ncmax = -(-nchunks // _NSUB)
    mesh = plsc.VectorSubcoreMesh(core_axis_name="c", subcore_axis_name="s")

    out_type = [jax.ShapeDtypeStruct((2, n_out, 64), jnp.float32)]
    scratch = [
        pltpu.VMEM((w,), jnp.int32),
        pltpu.VMEM((w,), jnp.int32),
        pltpu.VMEM((w, 64), jnp.float32),
        pltpu.VMEM_SHARED((n_out, 64), jnp.float32),
    ]
    if counts:
        out_type.append(jax.ShapeDtypeStruct((2, n_out, 16), jnp.float32))
        scratch += [
            pltpu.VMEM((w, 16), jnp.float32),
            pltpu.VMEM_SHARED((n_out, 16), jnp.float32),
        ]

    def body(table, isrc, idst, *refs):
        if counts:
            out, ocnt, src_v, dst_v, rows_v, acc, ones_v, cacc = refs
        else:
            out, src_v, dst_v, rows_v, acc = refs
            ocnt = ones_v = cacc = None
        c = lax.axis_index("c")
        s = lax.axis_index("s")

        _zero_rows(rows_v, 64, rc)
        if counts:
            _zero_rows(ones_v, 16, rc)

        def zero_body(i, _):
            j = i * _NSUB + s

            @pl.when(j < nchunks)
            def _():
                pltpu.sync_copy(rows_v.at[pl.ds(0, rc)],
                                acc.at[pl.ds(j * rc, rc)])
                if counts:
                    pltpu.sync_copy(ones_v.at[pl.ds(0, rc)],
                                    cacc.at[pl.ds(j * rc, rc)])
            return 0
        lax.fori_loop(0, ncmax, zero_body, 0)

        if counts:
            ov = jnp.ones((16,), jnp.float32)

            def fill_ones(i, _):
                ones_v[i, pl.ds(0, 16)] = ov
                return 0
            lax.fori_loop(0, w, fill_ones, 0)
        plsc.subcore_barrier()

        def window_body(i, _):
            wi = i * _NSUB + s

            @pl.when(wi < nw)
            def _():
                base = wi * w
                pltpu.sync_copy(idst.at[pl.ds(base, w)], dst_v)
                if gather:
                    pltpu.sync_copy(isrc.at[pl.ds(base, w)], src_v)
                    pltpu.sync_copy(table.at[c].at[src_v], rows_v)
                else:
                    pltpu.sync_copy(table.at[c].at[pl.ds(base, w)], rows_v)
                pltpu.sync_copy(rows_v, acc.at[dst_v], add=True)
                if counts:
                    @pl.when((wi % 2) == c)
                    def _():
                        pltpu.sync_copy(ones_v, cacc.at[dst_v], add=True)
            return 0
        lax.fori_loop(0, nwmax, window_body, 0)
        plsc.subcore_barrier()

        def wb_body(i, _):
            j = i * _NSUB + s

            @pl.when(j < nchunks)
            def _():
                pltpu.sync_copy(acc.at[pl.ds(j * rc, rc)],
                                out.at[c].at[pl.ds(j * rc, rc)])
                if counts:
                    pltpu.sync_copy(cacc.at[pl.ds(j * rc, rc)],
                                    ocnt.at[c].at[pl.ds(j * rc, rc)])
            return 0
        lax.fori_loop(0, ncmax, wb_body, 0)

    return pl.kernel(body, out_type=out_type, mesh=mesh,
                     scratch_types=scratch,
                     compiler_params=pltpu.CompilerParams(
                         use_tc_tiling_on_sc=False))


def _make_sc_edge_pipe(n_in, n_out, n_idx, w):
    nw = n_idx // w
    assert nw * w == n_idx and w % 8 == 0 and nw >= _NSUB
    trips = -(-nw // _NSUB)
    pairs = -(-trips // 2)
    rc = _rc_for(n_out, cap=min(w, 256))
    nchunks = n_out // rc
    ncmax = -(-nchunks // _NSUB)
    mesh = plsc.VectorSubcoreMesh(core_axis_name="c", subcore_axis_name="s")

    scratch = [
        pltpu.VMEM((w,), jnp.int32), pltpu.VMEM((w,), jnp.int32),
        pltpu.VMEM((w, 64), jnp.float32),
        pltpu.VMEM((w,), jnp.int32), pltpu.VMEM((w,), jnp.int32),
        pltpu.VMEM((w, 64), jnp.float32),
        pltpu.VMEM_SHARED((n_out, 64), jnp.float32),
        pltpu.SemaphoreType.DMA, pltpu.SemaphoreType.DMA,
        pltpu.SemaphoreType.DMA, pltpu.SemaphoreType.DMA,
    ]

    def body(table, eidx, out, src0, dst0, rows0, src1, dst1, rows1,
             acc, sem0, sem1, isem0, isem1):
        c = lax.axis_index("c")
        s = lax.axis_index("s")
        isrc = eidx.at[0]
        idst = eidx.at[1]

        _zero_rows(rows0, 64, rc)

        def zero_body(i, _):
            j = i * _NSUB + s

            @pl.when(j < nchunks)
            def _():
                pltpu.sync_copy(rows0.at[pl.ds(0, rc)],
                                acc.at[pl.ds(j * rc, rc)])
            return 0
        lax.fori_loop(0, ncmax, zero_body, 0)
        plsc.subcore_barrier()

        def wi_of(j):
            return j * _NSUB + s

        def idx_start(j, srcb, dstb, isem):
            base = wi_of(j) * w
            pltpu.async_copy(idst.at[pl.ds(base, w)], dstb, isem)
            pltpu.async_copy(isrc.at[pl.ds(base, w)], srcb, isem)

        def gather_start(j, srcb, dstb, rowsb, sem, isem):
            base = wi_of(j) * w
            pltpu.make_async_copy(idst.at[pl.ds(base, w)], dstb, isem).wait()
            pltpu.make_async_copy(isrc.at[pl.ds(base, w)], srcb, isem).wait()
            pltpu.async_copy(table.at[c].at[srcb], rowsb, sem)

        def wait_and_scatter(srcb, dstb, rowsb, sem):
            pltpu.make_async_copy(table.at[c].at[srcb], rowsb, sem).wait()
            pltpu.sync_copy(rowsb, acc.at[dstb], add=True)

        @pl.when(wi_of(0) < nw)
        def _():
            idx_start(0, src0, dst0, isem0)
            gather_start(0, src0, dst0, rows0, sem0, isem0)

        @pl.when(wi_of(1) < nw)
        def _():
            idx_start(1, src1, dst1, isem1)

        def pair_body(i, _):
            j0, j1, j2, j3 = 2 * i, 2 * i + 1, 2 * i + 2, 2 * i + 3

            @pl.when(wi_of(j1) < nw)
            def _():
                gather_start(j1, src1, dst1, rows1, sem1, isem1)

            @pl.when(wi_of(j0) < nw)
            def _():
                wait_and_scatter(src0, dst0, rows0, sem0)

            @pl.when(wi_of(j2) < nw)
            def _():
                idx_start(j2, src0, dst0, isem0)
                gather_start(j2, src0, dst0, rows0, sem0, isem0)

            @pl.when(wi_of(j1) < nw)
            def _():
                wait_and_scatter(src1, dst1, rows1, sem1)

            @pl.when(wi_of(j3) < nw)
            def _():
                idx_start(j3, src1, dst1, isem1)
            return 0
        lax.fori_loop(0, pairs, pair_body, 0)
        plsc.subcore_barrier()

        def wb_body(i, _):
            j = i * _NSUB + s

            @pl.when(j < nchunks)
            def _():
                pltpu.sync_copy(acc.at[pl.ds(j * rc, rc)],
                                out.at[c].at[pl.ds(j * rc, rc)])
            return 0
        lax.fori_loop(0, ncmax, wb_body, 0)

    return pl.kernel(body,
                     out_type=[jax.ShapeDtypeStruct((2, n_out, 64),
                                                    jnp.float32)],
                     mesh=mesh, scratch_types=scratch,
                     compiler_params=pltpu.CompilerParams(
                         use_tc_tiling_on_sc=False))


def _make_sc_edge_full(n, n_idx, w):
    nw = n_idx // w
    assert nw * w == n_idx and w % 8 == 0
    trips = -(-nw // (2 * _NSUB))
    pairs = -(-trips // 2)
    rc = _rc_for(n, cap=min(w, 256))
    nchunks = n // rc
    ncmax = -(-nchunks // _NSUB)
    mesh = plsc.VectorSubcoreMesh(core_axis_name="c", subcore_axis_name="s")

    scratch = [
        pltpu.VMEM((w,), jnp.int32), pltpu.VMEM((w,), jnp.int32),
        pltpu.VMEM((w, 128), jnp.float32),
        pltpu.VMEM((w,), jnp.int32), pltpu.VMEM((w,), jnp.int32),
        pltpu.VMEM((w, 128), jnp.float32),
        pltpu.VMEM_SHARED((n, 128), jnp.float32),
        pltpu.SemaphoreType.DMA, pltpu.SemaphoreType.DMA,
        pltpu.SemaphoreType.DMA, pltpu.SemaphoreType.DMA,
    ]

    def body(table, isrc, idst, out, src0, dst0, rows0, src1, dst1, rows1,
             acc, sem0, sem1, isem0, isem1):
        c = lax.axis_index("c")
        s = lax.axis_index("s")

        _zero_rows(rows0, 128, rc)

        def zero_body(i, _):
            j = i * _NSUB + s

            @pl.when(j < nchunks)
            def _():
                pltpu.sync_copy(rows0.at[pl.ds(0, rc)],
                                acc.at[pl.ds(j * rc, rc)])
            return 0
        lax.fori_loop(0, ncmax, zero_body, 0)
        plsc.subcore_barrier()

        def wi_of(j):
            return (j * _NSUB + s) * 2 + c

        def idx_start(j, srcb, dstb, isem):
            base = wi_of(j) * w
            pltpu.async_copy(idst.at[pl.ds(base, w)], dstb, isem)
            pltpu.async_copy(isrc.at[pl.ds(base, w)], srcb, isem)

        def gather_start(j, srcb, dstb, rowsb, sem, isem):
            base = wi_of(j) * w
            pltpu.make_async_copy(idst.at[pl.ds(base, w)], dstb, isem).wait()
            pltpu.make_async_copy(isrc.at[pl.ds(base, w)], srcb, isem).wait()
            pltpu.async_copy(table.at[srcb], rowsb, sem)

        def wait_and_scatter(srcb, dstb, rowsb, sem):
            pltpu.make_async_copy(table.at[srcb], rowsb, sem).wait()
            pltpu.sync_copy(rowsb, acc.at[dstb], add=True)

        @pl.when(wi_of(0) < nw)
        def _():
            idx_start(0, src0, dst0, isem0)
            gather_start(0, src0, dst0, rows0, sem0, isem0)

        @pl.when(wi_of(1) < nw)
        def _():
            idx_start(1, src1, dst1, isem1)

        def pair_body(i, _):
            j0, j1, j2, j3 = 2 * i, 2 * i + 1, 2 * i + 2, 2 * i + 3

            @pl.when(wi_of(j1) < nw)
            def _():
                gather_start(j1, src1, dst1, rows1, sem1, isem1)

            @pl.when(wi_of(j0) < nw)
            def _():
                wait_and_scatter(src0, dst0, rows0, sem0)

            @pl.when(wi_of(j2) < nw)
            def _():
                idx_start(j2, src0, dst0, isem0)
                gather_start(j2, src0, dst0, rows0, sem0, isem0)

            @pl.when(wi_of(j1) < nw)
            def _():
                wait_and_scatter(src1, dst1, rows1, sem1)

            @pl.when(wi_of(j3) < nw)
            def _():
                idx_start(j3, src1, dst1, isem1)
            return 0
        lax.fori_loop(0, pairs, pair_body, 0)
        plsc.subcore_barrier()

        def wb_body(i, _):
            j = i * _NSUB + s

            @pl.when(j < nchunks)
            def _():
                pltpu.sync_copy(acc.at[pl.ds(j * rc, rc)],
                                out.at[c].at[pl.ds(j * rc, rc)])
            return 0
        lax.fori_loop(0, ncmax, wb_body, 0)

    return pl.kernel(body,
                     out_type=[jax.ShapeDtypeStruct((2, n, 128),
                                                    jnp.float32)],
                     mesh=mesh, scratch_types=scratch,
                     compiler_params=pltpu.CompilerParams(
                         use_tc_tiling_on_sc=True))


def _make_sc_assign_pipe(n_in, n_out, n_idx, w):
    nw = n_idx // w
    assert nw * w == n_idx and w % 8 == 0 and nw >= _NSUB
    trips = -(-nw // _NSUB)
    pairs = -(-trips // 2)
    rc = _rc_for(n_out, cap=min(w, 256))
    nchunks = n_out // rc
    ncmax = -(-nchunks // _NSUB)
    mesh = plsc.VectorSubcoreMesh(core_axis_name="c", subcore_axis_name="s")

    scratch = [
        pltpu.VMEM((w,), jnp.int32), pltpu.VMEM((w,), jnp.int32),
        pltpu.VMEM((w, 64), jnp.float32),
        pltpu.VMEM((w,), jnp.int32), pltpu.VMEM((w,), jnp.int32),
        pltpu.VMEM((w, 64), jnp.float32),
        pltpu.VMEM((w, 16), jnp.float32),
        pltpu.VMEM_SHARED((n_out, 64), jnp.float32),
        pltpu.VMEM_SHARED((n_out, 16), jnp.float32),
        pltpu.SemaphoreType.DMA, pltpu.SemaphoreType.DMA,
        pltpu.SemaphoreType.DMA, pltpu.SemaphoreType.DMA,
    ]
    out_type = [jax.ShapeDtypeStruct((2, n_out, 64), jnp.float32),
                jax.ShapeDtypeStruct((2, n_out, 16), jnp.float32)]

    def body(table, isrc, idst, out, ocnt, src0, dst0, rows0, src1, dst1,
             rows1, ones_v, acc, cacc, sem0, sem1, isem0, isem1):
        c = lax.axis_index("c")
        s = lax.axis_index("s")

        _zero_rows(rows0, 64, rc)
        _zero_rows(ones_v, 16, rc)

        def zero_body(i, _):
            j = i * _NSUB + s

            @pl.when(j < nchunks)
            def _():
                pltpu.sync_copy(rows0.at[pl.ds(0, rc)],
                                acc.at[pl.ds(j * rc, rc)])
                pltpu.sync_copy(ones_v.at[pl.ds(0, rc)],
                                cacc.at[pl.ds(j * rc, rc)])
            return 0
        lax.fori_loop(0, ncmax, zero_body, 0)

        ov = jnp.ones((16,), jnp.float32)

        def fill_ones(i, _):
            ones_v[i, pl.ds(0, 16)] = ov
            return 0
        lax.fori_loop(0, w, fill_ones, 0)
        plsc.subcore_barrier()

        def wi_of(j):
            return j * _NSUB + s

        def idx_start(j, srcb, dstb, isem):
            base = wi_of(j) * w
            pltpu.async_copy(idst.at[pl.ds(base, w)], dstb, isem)
            pltpu.async_copy(isrc.at[pl.ds(base, w)], srcb, isem)

        def gather_start(j, srcb, dstb, rowsb, sem, isem):
            base = wi_of(j) * w
            pltpu.make_async_copy(idst.at[pl.ds(base, w)], dstb, isem).wait()
            pltpu.make_async_copy(isrc.at[pl.ds(base, w)], srcb, isem).wait()
            pltpu.async_copy(table.at[c].at[srcb], rowsb, sem)

        def wait_and_scatter(j, srcb, dstb, rowsb, sem):
            pltpu.make_async_copy(table.at[c].at[srcb], rowsb, sem).wait()
            pltpu.sync_copy(rowsb, acc.at[dstb], add=True)

            @pl.when((wi_of(j) % 2) == c)
            def _():
                pltpu.sync_copy(ones_v, cacc.at[dstb], add=True)

        @pl.when(wi_of(0) < nw)
        def _():
            idx_start(0, src0, dst0, isem0)
            gather_start(0, src0, dst0, rows0, sem0, isem0)

        @pl.when(wi_of(1) < nw)
        def _():
            idx_start(1, src1, dst1, isem1)

        def pair_body(i, _):
            j0, j1, j2, j3 = 2 * i, 2 * i + 1, 2 * i + 2, 2 * i + 3

            @pl.when(wi_of(j1) < nw)
            def _():
                gather_start(j1, src1, dst1, rows1, sem1, isem1)

            @pl.when(wi_of(j0) < nw)
            def _():
                wait_and_scatter(j0, src0, dst0, rows0, sem0)

            @pl.when(wi_of(j2) < nw)
            def _():
                idx_start(j2, src0, dst0, isem0)
                gather_start(j2, src0, dst0, rows0, sem0, isem0)

            @pl.when(wi_of(j1) < nw)
            def _():
                wait_and_scatter(j1, src1, dst1, rows1, sem1)

            @pl.when(wi_of(j3) < nw)
            def _():
                idx_start(j3, src1, dst1, isem1)
            return 0
        lax.fori_loop(0, pairs, pair_body, 0)
        plsc.subcore_barrier()

        def wb_body(i, _):
            j = i * _NSUB + s

            @pl.when(j < nchunks)
            def _():
                pltpu.sync_copy(acc.at[pl.ds(j * rc, rc)],
                                out.at[c].at[pl.ds(j * rc, rc)])
                pltpu.sync_copy(cacc.at[pl.ds(j * rc, rc)],
                                ocnt.at[c].at[pl.ds(j * rc, rc)])
            return 0
        lax.fori_loop(0, ncmax, wb_body, 0)

    return pl.kernel(body, out_type=out_type, mesh=mesh,
                     scratch_types=scratch,
                     compiler_params=pltpu.CompilerParams(
                         use_tc_tiling_on_sc=False))


_sc_edge1 = _make_sc_edge_full(_N, _E, w=160)
_sc_edge2 = _make_sc_edge_pipe(_N2, _N2, _E2, w=320)
_sc_assign = _make_sc_assign_pipe(_N, _N2, _A, w=160)
_sc_pool1 = _make_sc_segsum(_N, _B, _N, gather=False, counts=True, w=400)
_sc_pool2 = _make_sc_segsum(_N2, _B, _N2, gather=False, counts=True, w=400)


def kernel(x, iso_type_2, Wi_r, Wi_n, bi, W00_r, W00_n, b00, W01_r, W01_n, b01,
           W10_r, W10_n, b10, W11_r, W11_n, b11,
           fc0_w, fc0_b, fc1_w, fc1_b, fc2_w, fc2_b,
           edge_index, edge_index_2, assignment_index_2, batch, batch_2):
    src, dst = edge_index[0], edge_index[1]
    asrc, adst = assignment_index_2[0], assignment_index_2[1]

    (s0,) = _sc_edge1(x, src, dst)
    h1 = _tc_step1(x, s0, bi, Wi_r, Wi_n)
    (s1,) = _sc_edge1(h1, src, dst)
    h2 = _tc_step1(h1, s1, b00, W00_r, W00_n)
    (s2,) = _sc_edge1(h2, src, dst)
    hs = _tc_fin1(h2, s2, b01, W01_r, W01_n)

    p1s, p1c = _sc_pool1(hs, batch, batch)

    asum, acnt = _sc_assign(hs, asrc, adst)
    r3, t3 = _tc_assign(asum, acnt, iso_type_2, W10_r, W10_n)
    (s3,) = _sc_edge2(t3, edge_index_2)
    r4, t4 = _tc_step(r3, s3, b10, W11_r, W11_n)
    (s4,) = _sc_edge2(t4, edge_index_2)
    h2s = _tc_fin(r4, s4, b11)

    p2s, p2c = _sc_pool2(h2s, batch_2, batch_2)

    return _tc_head(p1s, p1c, p2s, p2c, fc0_w, fc0_b, fc1_w, fc1_b,
                    fc2_w, fc2_b)
